# Initial kernel scaffold; baseline (speedup 1.0000x reference)
import jax
import jax.numpy as jnp
from jax.experimental import pallas as pl


def kernel(pos, seq, ori, domain, seq_emb, params, x, edge_index, batch):
    raise NotImplementedError("write your pallas kernel here")


# speedup vs baseline: 4.5927x; 4.5927x over previous
import functools

import jax
import jax.numpy as jnp
from jax import lax
from jax.experimental import pallas as pl
from jax.experimental.pallas import tpu as pltpu
from jax.experimental.pallas import tpu_sc as plsc

F32 = jnp.float32
I32 = jnp.int32

N0 = 50000
E = 800000
EP = 819200
EC = 3200
NEB = EP // EC
NCH = 1600
LVL_N = [50000, 25000, 12500, 6250]
LVL_NP = [51200, 25600, 12800, 6400]
B = 64
SEQ_L = 5.0
IO_CH = [(16, 32), (32, 32), (32, 64), (64, 64),
         (64, 128), (128, 128), (128, 256), (256, 256)]


def _embed_call(x2, emb32, np0):
    def body(x_ref, emb_ref, out_ref):
        lane = lax.broadcasted_iota(I32, (NCH, 32), 1)
        oh = (lane == x_ref[...]).astype(F32)
        out_ref[...] = jnp.dot(oh, emb_ref[...], preferred_element_type=F32)

    return pl.pallas_call(
        body,
        grid=(np0 // NCH,),
        in_specs=[
            pl.BlockSpec((NCH, 1), lambda i: (i, 0)),
            pl.BlockSpec((32, 16), lambda i: (0, 0)),
        ],
        out_specs=pl.BlockSpec((NCH, 16), lambda i: (i, 0)),
        out_shape=jax.ShapeDtypeStruct((np0, 16), F32),
    )(x2, emb32)


def _msg_a_call(gs, gd, s2, d2, hs, k1a, k2a, k1b, k2b, lvl):
    ciA = k2a.shape[1]
    ciB = k2b.shape[1]
    scale = float(2 ** lvl) / SEQ_L

    def body(gs_ref, gd_ref, s_ref, d_ref, hs_ref, k1a_ref, k2a_ref,
             k1b_ref, k2b_ref, msg_ref, kb_ref):
        gsv = gs_ref[...]
        gdv = gd_ref[...]
        sl = jnp.right_shift(s_ref[...], lvl)
        dl = jnp.right_shift(d_ref[...], lvl)
        rel = (dl - sl).astype(F32) * scale
        lane = lax.broadcasted_iota(I32, (1, 16), 1)
        base = jnp.where(lane < 3, gdv - gsv, gsv * gdv)
        geo = base + rel * (lane == 3).astype(F32)
        ka = jnp.maximum(
            jnp.dot(geo, k1a_ref[...], preferred_element_type=F32), 0.0)
        kern_a = jnp.dot(ka, k2a_ref[...], preferred_element_type=F32)
        msg_ref[...] = hs_ref[...] * kern_a
        kb = jnp.maximum(
            jnp.dot(geo, k1b_ref[...], preferred_element_type=F32), 0.0)
        kb_ref[...] = jnp.dot(kb, k2b_ref[...], preferred_element_type=F32)

    return pl.pallas_call(
        body,
        grid=(NEB,),
        in_specs=[
            pl.BlockSpec((EC, 16), lambda i: (i, 0)),
            pl.BlockSpec((EC, 16), lambda i: (i, 0)),
            pl.BlockSpec((EC, 1), lambda i: (i, 0)),
            pl.BlockSpec((EC, 1), lambda i: (i, 0)),
            pl.BlockSpec((EC, ciA), lambda i: (i, 0)),
            pl.BlockSpec((16, 24), lambda i: (0, 0)),
            pl.BlockSpec((24, ciA), lambda i: (0, 0)),
            pl.BlockSpec((16, 24), lambda i: (0, 0)),
            pl.BlockSpec((24, ciB), lambda i: (0, 0)),
        ],
        out_specs=[
            pl.BlockSpec((EC, ciA), lambda i: (i, 0)),
            pl.BlockSpec((EC, ciB), lambda i: (i, 0)),
        ],
        out_shape=[
            jax.ShapeDtypeStruct((EP, ciA), F32),
            jax.ShapeDtypeStruct((EP, ciB), F32),
        ],
    )(gs, gd, s2, d2, hs, k1a, k2a, k1b, k2b)


def _mult_call(hs, kern):
    ci = hs.shape[1]

    def body(hs_ref, k_ref, out_ref):
        out_ref[...] = hs_ref[...] * k_ref[...]

    return pl.pallas_call(
        body,
        grid=(NEB,),
        in_specs=[
            pl.BlockSpec((EC, ci), lambda i: (i, 0)),
            pl.BlockSpec((EC, ci), lambda i: (i, 0)),
        ],
        out_specs=pl.BlockSpec((EC, ci), lambda i: (i, 0)),
        out_shape=jax.ShapeDtypeStruct((EP, ci), F32),
    )(hs, kern)


def _node_update(aggp, degp, h, lin, res, n_l, np_l):
    ci = lin.shape[0]
    co = lin.shape[1]
    nsteps = np_l // NCH
    inv_n = 1.0 / float(n_l)

    def body1(p0_ref, p1_ref, d0_ref, d1_ref, lin_ref, z_ref, st_ref):
        i = pl.program_id(0)
        deg = d0_ref[...][:, 0:1] + d1_ref[...][:, 0:1]
        agg = (p0_ref[...] + p1_ref[...]) / jnp.maximum(deg, 1.0)
        z = jnp.dot(agg, lin_ref[...], preferred_element_type=F32)
        z_ref[...] = z
        rid = i * NCH + lax.broadcasted_iota(I32, (NCH, 1), 0)
        m = (rid < n_l).astype(F32)
        zm = z * m
        s1 = jnp.sum(zm, axis=0, keepdims=True)
        s2 = jnp.sum(zm * z, axis=0, keepdims=True)

        @pl.when(i == 0)
        def _():
            st_ref[...] = jnp.zeros_like(st_ref)

        st_ref[0:1, :] += s1
        st_ref[1:2, :] += s2

    z, st = pl.pallas_call(
        body1,
        grid=(nsteps,),
        in_specs=[
            pl.BlockSpec((NCH, ci), lambda i: (i, 0)),
            pl.BlockSpec((NCH, ci), lambda i: (i, 0)),
            pl.BlockSpec((NCH, 16), lambda i: (i, 0)),
            pl.BlockSpec((NCH, 16), lambda i: (i, 0)),
            pl.BlockSpec((ci, co), lambda i: (0, 0)),
        ],
        out_specs=[
            pl.BlockSpec((NCH, co), lambda i: (i, 0)),
            pl.BlockSpec((8, co), lambda i: (0, 0)),
        ],
        out_shape=[
            jax.ShapeDtypeStruct((np_l, co), F32),
            jax.ShapeDtypeStruct((8, co), F32),
        ],
    )(aggp[0], aggp[1], degp[0], degp[1], lin)

    def body2(z_ref, st_ref, h_ref, res_ref, out_ref):
        mean = st_ref[0:1, :] * inv_n
        var = st_ref[1:2, :] * inv_n - mean * mean
        std = jnp.sqrt(jnp.maximum(var, 0.0))
        zn = (z_ref[...] - mean) / (std + 1e-5)
        out_ref[...] = jnp.maximum(zn, 0.0) + jnp.dot(
            h_ref[...], res_ref[...], preferred_element_type=F32)

    return pl.pallas_call(
        body2,
        grid=(nsteps,),
        in_specs=[
            pl.BlockSpec((NCH, co), lambda i: (i, 0)),
            pl.BlockSpec((8, co), lambda i: (0, 0)),
            pl.BlockSpec((NCH, ci), lambda i: (i, 0)),
            pl.BlockSpec((ci, co), lambda i: (0, 0)),
        ],
        out_specs=pl.BlockSpec((NCH, co), lambda i: (i, 0)),
        out_shape=jax.ShapeDtypeStruct((np_l, co), F32),
    )(z, st, h, res)


def _pool_call(a3):
    m = a3.shape[0]
    d = a3.shape[2]

    def body(a_ref, out_ref):
        out_ref[...] = (a_ref[:, 0, :] + a_ref[:, 1, :]) * 0.5

    return pl.pallas_call(
        body,
        grid=(m // NCH,),
        in_specs=[pl.BlockSpec((NCH, 2, d), lambda i: (i, 0, 0))],
        out_specs=pl.BlockSpec((NCH, d), lambda i: (i, 0)),
        out_shape=jax.ShapeDtypeStruct((m, d), F32),
    )(a3)


def _classifier_call(g0, g1, c0, c1, seq_emb, domain, ws, wq, wd,
                     wc1a, wc1b, wc1c, wc2):
    def body(g0_ref, g1_ref, c0_ref, c1_ref, se_ref, dom_ref, ws_ref,
             wq_ref, wd_ref, a_ref, b_ref, c_ref, w2_ref, out_ref):
        cnt = c0_ref[...][:, 0:1] + c1_ref[...][:, 0:1]
        g = (g0_ref[...] + g1_ref[...]) / jnp.maximum(cnt, 1.0)
        struct = jnp.dot(g, ws_ref[...], preferred_element_type=F32)
        seqf = jnp.dot(se_ref[...], wq_ref[...], preferred_element_type=F32)
        dom = dom_ref[...]
        mask = jnp.sum(dom, axis=1, keepdims=True) != 0.0
        domf = jnp.where(mask,
                         jnp.dot(dom, wd_ref[...], preferred_element_type=F32),
                         0.0)
        hid = (jnp.dot(struct, a_ref[...], preferred_element_type=F32)
               + jnp.dot(seqf, b_ref[...], preferred_element_type=F32)
               + jnp.dot(domf, c_ref[...], preferred_element_type=F32))
        mean = jnp.mean(hid, axis=0, keepdims=True)
        var = jnp.mean(hid * hid, axis=0, keepdims=True) - mean * mean
        std = jnp.sqrt(jnp.maximum(var, 0.0))
        hid = jnp.maximum((hid - mean) / (std + 1e-5), 0.0)
        out_ref[...] = jnp.dot(hid, w2_ref[...], preferred_element_type=F32)

    nc = wc2.shape[1]
    return pl.pallas_call(
        body,
        out_shape=jax.ShapeDtypeStruct((B, nc), F32),
    )(g0, g1, c0, c1, seq_emb, domain, ws, wq, wd, wc1a, wc1b, wc1c, wc2)


NW = 32
SCCH = 128


def _shift_idx(idx_v, lvl, ch):
    if lvl:
        for t in range(ch // 16):
            sl = pl.ds(t * 16, 16)
            idx_v[sl] = jnp.right_shift(idx_v[sl], lvl)


def _sc_gather(table, idx, lvl):
    d = table.shape[1]
    ep = idx.shape[0]
    per_w = ep // NW
    nch = per_w // SCCH
    mesh = plsc.VectorSubcoreMesh(core_axis_name="c", subcore_axis_name="s")

    @functools.partial(
        pl.kernel,
        out_type=jax.ShapeDtypeStruct((ep, d), F32),
        mesh=mesh,
        compiler_params=pltpu.CompilerParams(use_tc_tiling_on_sc=False),
        scratch_types=[
            pltpu.VMEM((SCCH,), I32),
            pltpu.VMEM((SCCH, d), F32),
            pltpu.SemaphoreType.DMA,
        ],
    )
    def k(table_hbm, idx_hbm, out_hbm, idx_v, rows_v, sem):
        wid = lax.axis_index("s") * 2 + lax.axis_index("c")
        base_w = wid * per_w

        def body(j, carry):
            base = base_w + j * SCCH
            pltpu.sync_copy(idx_hbm.at[pl.ds(base, SCCH)], idx_v)
            _shift_idx(idx_v, lvl, SCCH)
            pltpu.async_copy(table_hbm.at[idx_v], rows_v, sem).wait()
            pltpu.sync_copy(rows_v, out_hbm.at[pl.ds(base, SCCH)])
            return carry

        lax.fori_loop(0, nch, body, 0)

    return k(table, idx)


def _sc_scatter(rows, idx, lvl, np_l, d, ones_mode=False, col0=0, dfull=None):
    if dfull is None:
        dfull = d
    ep = idx.shape[0]
    per_w = ep // NW
    nch = per_w // SCCH
    rows_pt = np_l // 16
    zr = 8
    mesh = plsc.VectorSubcoreMesh(core_axis_name="c", subcore_axis_name="s")

    scratch = [
        pltpu.VMEM((SCCH,), I32),
        pltpu.VMEM((SCCH, d), F32),
        pltpu.VMEM((zr, d), F32),
        pltpu.VMEM_SHARED((np_l, d), F32),
    ]

    def body_common(rows_hbm, idx_hbm, out_hbm, idx_v, rows_v, zbuf, acc):
        cid = lax.axis_index("c")
        sid = lax.axis_index("s")
        wid = sid * 2 + cid
        for r in range(zr):
            for t in range(d // 16):
                zbuf[r, pl.ds(t * 16, 16)] = jnp.zeros((16,), F32)
        r0 = sid * rows_pt

        def zb(j, carry):
            pltpu.sync_copy(zbuf, acc.at[pl.ds(r0 + j * zr, zr)])
            return carry

        lax.fori_loop(0, rows_pt // zr, zb, 0)

        if ones_mode:
            for r in range(SCCH):
                for t in range(d // 16):
                    rows_v[r, pl.ds(t * 16, 16)] = jnp.ones((16,), F32)

        plsc.subcore_barrier()
        base_w = wid * per_w

        def body(j, carry):
            base = base_w + j * SCCH
            pltpu.sync_copy(idx_hbm.at[pl.ds(base, SCCH)], idx_v)
            _shift_idx(idx_v, lvl, SCCH)
            if not ones_mode:
                if d == dfull:
                    pltpu.sync_copy(rows_hbm.at[pl.ds(base, SCCH)], rows_v)
                else:
                    pltpu.sync_copy(
                        rows_hbm.at[pl.ds(base, SCCH), pl.ds(col0, d)],
                        rows_v)
            pltpu.sync_copy(rows_v, acc.at[idx_v], add=True)
            return carry

        lax.fori_loop(0, nch, body, 0)
        plsc.subcore_barrier()

        def rb(j, carry):
            r = r0 + j * zr
            pltpu.sync_copy(acc.at[pl.ds(r, zr)], zbuf)
            pltpu.sync_copy(zbuf, out_hbm.at[cid].at[pl.ds(r, zr)])
            return carry

        lax.fori_loop(0, rows_pt // zr, rb, 0)

    if ones_mode:
        @functools.partial(
            pl.kernel,
            out_type=jax.ShapeDtypeStruct((2, np_l, d), F32),
            mesh=mesh, scratch_types=scratch,
            compiler_params=pltpu.CompilerParams(use_tc_tiling_on_sc=False),
        )
        def k1(idx_hbm, out_hbm, idx_v, rows_v, zbuf, acc):
            body_common(None, idx_hbm, out_hbm, idx_v, rows_v, zbuf, acc)

        return k1(idx)

    @functools.partial(
        pl.kernel,
        out_type=jax.ShapeDtypeStruct((2, np_l, d), F32),
        mesh=mesh, scratch_types=scratch,
        compiler_params=pltpu.CompilerParams(use_tc_tiling_on_sc=False),
    )
    def k2(rows_hbm, idx_hbm, out_hbm, idx_v, rows_v, zbuf, acc):
        body_common(rows_hbm, idx_hbm, out_hbm, idx_v, rows_v, zbuf, acc)

    return k2(rows, idx)


def _sc_scatter_add(rows, idx, lvl, np_l):
    d = rows.shape[1]
    if np_l * d * 4 > 5_000_000 and d > 128:
        parts = [_sc_scatter(rows, idx, lvl, np_l, 128, col0=c, dfull=d)
                 for c in range(0, d, 128)]
        return jnp.concatenate(parts, axis=2)
    return _sc_scatter(rows, idx, lvl, np_l, d)


def _sc_scatter_ones(idx, lvl, np_l):
    return _sc_scatter(None, idx, lvl, np_l, 16, ones_mode=True)


def kernel(pos, seq, ori, domain, seq_emb, params, x, edge_index, batch):
    np0 = LVL_NP[0]
    src = edge_index[0].astype(I32)
    dst = edge_index[1].astype(I32)
    src_p = jnp.concatenate([src, jnp.zeros((EP - E,), I32)])
    dst_p = jnp.concatenate([dst, jnp.full((EP - E,), N0, I32)])
    s2 = src_p.reshape(EP, 1)
    d2 = dst_p.reshape(EP, 1)

    g_tab = jnp.concatenate(
        [pos, jnp.zeros((N0, 1), F32), ori.reshape(N0, 9),
         jnp.zeros((N0, 3), F32)], axis=1)
    g_tab = jnp.pad(g_tab, ((0, np0 - N0), (0, 0)))

    x2 = jnp.pad(x.astype(I32), (0, np0 - N0),
                 constant_values=31).reshape(np0, 1)
    emb32 = jnp.pad(params["emb"], ((0, 11), (0, 0)))
    h = _embed_call(x2, emb32, np0)

    g0 = g1 = c0 = c1 = None
    for lvl in range(4):
        n_l = LVL_N[lvl]
        np_l = LVL_NP[lvl]
        iA, iB = 2 * lvl, 2 * lvl + 1
        k1a = jnp.concatenate([params[f"b{iA}_k1"], jnp.zeros((3, 24), F32)])
        k1b = jnp.concatenate([params[f"b{iB}_k1"], jnp.zeros((3, 24), F32)])
        k2a = params[f"b{iA}_k2"]
        k2b = params[f"b{iB}_k2"]

        gs = _sc_gather(g_tab, src_p, lvl)
        gd = _sc_gather(g_tab, dst_p, lvl)
        degp = _sc_scatter_ones(dst_p, lvl, np_l)

        hs_a = _sc_gather(h, src_p, lvl)
        msg_a, kern_b = _msg_a_call(gs, gd, s2, d2, hs_a,
                                    k1a, k2a, k1b, k2b, lvl)
        agg_a = _sc_scatter_add(msg_a, dst_p, lvl, np_l)
        h = _node_update(agg_a, degp, h, params[f"b{iA}_lin"],
                         params[f"b{iA}_res"], n_l, np_l)

        hs_b = _sc_gather(h, src_p, lvl)
        msg_b = _mult_call(hs_b, kern_b)
        agg_b = _sc_scatter_add(msg_b, dst_p, lvl, np_l)
        h = _node_update(agg_b, degp, h, params[f"b{iB}_lin"],
                         params[f"b{iB}_res"], n_l, np_l)

        if lvl < 3:
            m = np_l // 2
            g_tab = _pool_call(g_tab.reshape(m, 2, 16))
            h = _pool_call(h.reshape(m, 2, h.shape[1]))

    batch_l = batch[::8].astype(I32)
    epg = 8192
    batch_p = jnp.pad(batch_l, (0, epg - LVL_N[3]), constant_values=B)
    h_p = jnp.pad(h, ((0, epg - LVL_NP[3]), (0, 0)))
    gpart = _sc_scatter_add(h_p, batch_p, 0, 128)
    cpart = _sc_scatter_ones(batch_p, 0, 128)

    wc1 = params["Wc1"]
    out = _classifier_call(
        gpart[0, :B], gpart[1, :B], cpart[0, :B], cpart[1, :B],
        seq_emb, domain, params["Ws"], params["Wq"], params["Wd"],
        wc1[0:256], wc1[256:512], wc1[512:768], params["Wc2"])
    return out


# speedup vs baseline: 5.0025x; 1.0892x over previous
import functools

import jax
import jax.numpy as jnp
from jax import lax
from jax.experimental import pallas as pl
from jax.experimental.pallas import tpu as pltpu
from jax.experimental.pallas import tpu_sc as plsc

F32 = jnp.float32
I32 = jnp.int32

N0 = 50000
E = 800000
EP = 819200
EC = 3200
NEB = EP // EC
NCH = 1600
LVL_N = [50000, 25000, 12500, 6250]
LVL_NP = [51200, 25600, 12800, 6400]
B = 64
SEQ_L = 5.0
IO_CH = [(16, 32), (32, 32), (32, 64), (64, 64),
         (64, 128), (128, 128), (128, 256), (256, 256)]


def _embed_call(x2, emb32, np0):
    def body(x_ref, emb_ref, out_ref):
        lane = lax.broadcasted_iota(I32, (NCH, 32), 1)
        oh = (lane == x_ref[...]).astype(F32)
        out_ref[...] = jnp.dot(oh, emb_ref[...], preferred_element_type=F32)

    return pl.pallas_call(
        body,
        grid=(np0 // NCH,),
        in_specs=[
            pl.BlockSpec((NCH, 1), lambda i: (i, 0)),
            pl.BlockSpec((32, 16), lambda i: (0, 0)),
        ],
        out_specs=pl.BlockSpec((NCH, 16), lambda i: (i, 0)),
        out_shape=jax.ShapeDtypeStruct((np0, 16), F32),
    )(x2, emb32)


def _msg_a_call(gs, gd, s2, d2, hs, k1a, k2a, k1b, k2b, lvl):
    ciA = k2a.shape[1]
    ciB = k2b.shape[1]
    scale = float(2 ** lvl) / SEQ_L

    def body(gs_ref, gd_ref, s_ref, d_ref, hs_ref, k1a_ref, k2a_ref,
             k1b_ref, k2b_ref, msg_ref, kb_ref):
        gsv = gs_ref[...]
        gdv = gd_ref[...]
        sl = jnp.right_shift(s_ref[...], lvl)
        dl = jnp.right_shift(d_ref[...], lvl)
        rel = (dl - sl).astype(F32) * scale
        lane = lax.broadcasted_iota(I32, (1, 16), 1)
        base = jnp.where(lane < 3, gdv - gsv, gsv * gdv)
        geo = base + rel * (lane == 3).astype(F32)
        ka = jnp.maximum(
            jnp.dot(geo, k1a_ref[...], preferred_element_type=F32), 0.0)
        kern_a = jnp.dot(ka, k2a_ref[...], preferred_element_type=F32)
        msg_ref[...] = hs_ref[...] * kern_a
        kb = jnp.maximum(
            jnp.dot(geo, k1b_ref[...], preferred_element_type=F32), 0.0)
        kb_ref[...] = jnp.dot(kb, k2b_ref[...], preferred_element_type=F32)

    return pl.pallas_call(
        body,
        grid=(NEB,),
        in_specs=[
            pl.BlockSpec((EC, 16), lambda i: (i, 0)),
            pl.BlockSpec((EC, 16), lambda i: (i, 0)),
            pl.BlockSpec((EC, 1), lambda i: (i, 0)),
            pl.BlockSpec((EC, 1), lambda i: (i, 0)),
            pl.BlockSpec((EC, ciA), lambda i: (i, 0)),
            pl.BlockSpec((16, 24), lambda i: (0, 0)),
            pl.BlockSpec((24, ciA), lambda i: (0, 0)),
            pl.BlockSpec((16, 24), lambda i: (0, 0)),
            pl.BlockSpec((24, ciB), lambda i: (0, 0)),
        ],
        out_specs=[
            pl.BlockSpec((EC, ciA), lambda i: (i, 0)),
            pl.BlockSpec((EC, ciB), lambda i: (i, 0)),
        ],
        out_shape=[
            jax.ShapeDtypeStruct((EP, ciA), F32),
            jax.ShapeDtypeStruct((EP, ciB), F32),
        ],
    )(gs, gd, s2, d2, hs, k1a, k2a, k1b, k2b)


def _mult_call(hs, kern):
    ci = hs.shape[1]

    def body(hs_ref, k_ref, out_ref):
        out_ref[...] = hs_ref[...] * k_ref[...]

    return pl.pallas_call(
        body,
        grid=(NEB,),
        in_specs=[
            pl.BlockSpec((EC, ci), lambda i: (i, 0)),
            pl.BlockSpec((EC, ci), lambda i: (i, 0)),
        ],
        out_specs=pl.BlockSpec((EC, ci), lambda i: (i, 0)),
        out_shape=jax.ShapeDtypeStruct((EP, ci), F32),
    )(hs, kern)


def _node_update(aggp, degp, h, lin, res, n_l, np_l):
    ci = lin.shape[0]
    co = lin.shape[1]
    nsteps = np_l // NCH
    inv_n = 1.0 / float(n_l)

    def body1(p0_ref, p1_ref, d0_ref, d1_ref, lin_ref, z_ref, st_ref):
        i = pl.program_id(0)
        deg = d0_ref[...][:, 0:1] + d1_ref[...][:, 0:1]
        agg = (p0_ref[...] + p1_ref[...]) / jnp.maximum(deg, 1.0)
        z = jnp.dot(agg, lin_ref[...], preferred_element_type=F32)
        z_ref[...] = z
        rid = i * NCH + lax.broadcasted_iota(I32, (NCH, 1), 0)
        m = (rid < n_l).astype(F32)
        zm = z * m
        s1 = jnp.sum(zm, axis=0, keepdims=True)
        s2 = jnp.sum(zm * z, axis=0, keepdims=True)

        @pl.when(i == 0)
        def _():
            st_ref[...] = jnp.zeros_like(st_ref)

        st_ref[0:1, :] += s1
        st_ref[1:2, :] += s2

    z, st = pl.pallas_call(
        body1,
        grid=(nsteps,),
        in_specs=[
            pl.BlockSpec((NCH, ci), lambda i: (i, 0)),
            pl.BlockSpec((NCH, ci), lambda i: (i, 0)),
            pl.BlockSpec((NCH, 16), lambda i: (i, 0)),
            pl.BlockSpec((NCH, 16), lambda i: (i, 0)),
            pl.BlockSpec((ci, co), lambda i: (0, 0)),
        ],
        out_specs=[
            pl.BlockSpec((NCH, co), lambda i: (i, 0)),
            pl.BlockSpec((8, co), lambda i: (0, 0)),
        ],
        out_shape=[
            jax.ShapeDtypeStruct((np_l, co), F32),
            jax.ShapeDtypeStruct((8, co), F32),
        ],
    )(aggp[0], aggp[1], degp[0], degp[1], lin)

    def body2(z_ref, st_ref, h_ref, res_ref, out_ref):
        mean = st_ref[0:1, :] * inv_n
        var = st_ref[1:2, :] * inv_n - mean * mean
        std = jnp.sqrt(jnp.maximum(var, 0.0))
        zn = (z_ref[...] - mean) / (std + 1e-5)
        out_ref[...] = jnp.maximum(zn, 0.0) + jnp.dot(
            h_ref[...], res_ref[...], preferred_element_type=F32)

    return pl.pallas_call(
        body2,
        grid=(nsteps,),
        in_specs=[
            pl.BlockSpec((NCH, co), lambda i: (i, 0)),
            pl.BlockSpec((8, co), lambda i: (0, 0)),
            pl.BlockSpec((NCH, ci), lambda i: (i, 0)),
            pl.BlockSpec((ci, co), lambda i: (0, 0)),
        ],
        out_specs=pl.BlockSpec((NCH, co), lambda i: (i, 0)),
        out_shape=jax.ShapeDtypeStruct((np_l, co), F32),
    )(z, st, h, res)


def _pool_call(a3):
    m = a3.shape[0]
    d = a3.shape[2]

    def body(a_ref, out_ref):
        out_ref[...] = (a_ref[:, 0, :] + a_ref[:, 1, :]) * 0.5

    return pl.pallas_call(
        body,
        grid=(m // NCH,),
        in_specs=[pl.BlockSpec((NCH, 2, d), lambda i: (i, 0, 0))],
        out_specs=pl.BlockSpec((NCH, d), lambda i: (i, 0)),
        out_shape=jax.ShapeDtypeStruct((m, d), F32),
    )(a3)


def _classifier_call(g0, g1, c0, c1, seq_emb, domain, ws, wq, wd,
                     wc1a, wc1b, wc1c, wc2):
    def body(g0_ref, g1_ref, c0_ref, c1_ref, se_ref, dom_ref, ws_ref,
             wq_ref, wd_ref, a_ref, b_ref, c_ref, w2_ref, out_ref):
        cnt = c0_ref[...][:, 0:1] + c1_ref[...][:, 0:1]
        g = (g0_ref[...] + g1_ref[...]) / jnp.maximum(cnt, 1.0)
        struct = jnp.dot(g, ws_ref[...], preferred_element_type=F32)
        seqf = jnp.dot(se_ref[...], wq_ref[...], preferred_element_type=F32)
        dom = dom_ref[...]
        mask = jnp.sum(dom, axis=1, keepdims=True) != 0.0
        domf = jnp.where(mask,
                         jnp.dot(dom, wd_ref[...], preferred_element_type=F32),
                         0.0)
        hid = (jnp.dot(struct, a_ref[...], preferred_element_type=F32)
               + jnp.dot(seqf, b_ref[...], preferred_element_type=F32)
               + jnp.dot(domf, c_ref[...], preferred_element_type=F32))
        mean = jnp.mean(hid, axis=0, keepdims=True)
        var = jnp.mean(hid * hid, axis=0, keepdims=True) - mean * mean
        std = jnp.sqrt(jnp.maximum(var, 0.0))
        hid = jnp.maximum((hid - mean) / (std + 1e-5), 0.0)
        out_ref[...] = jnp.dot(hid, w2_ref[...], preferred_element_type=F32)

    nc = wc2.shape[1]
    return pl.pallas_call(
        body,
        out_shape=jax.ShapeDtypeStruct((B, nc), F32),
    )(g0, g1, c0, c1, seq_emb, domain, ws, wq, wd, wc1a, wc1b, wc1c, wc2)


NW = 32
SCCH = 128


def _shift_idx2(idx_v, lvl, g):
    if lvl:
        for r in range(g):
            for t in range(SCCH // 16):
                sl = pl.ds(t * 16, 16)
                idx_v[r, sl] = jnp.right_shift(idx_v[r, sl], lvl)


def _grp(d):
    return max(1, min(8, 262144 // (SCCH * d * 4)))


def _sc_gather(table, idx2, lvl):
    d = table.shape[1]
    ep = idx2.shape[0] * SCCH
    per_w = ep // NW
    g = _grp(d)
    ngrp = per_w // (SCCH * g)
    mesh = plsc.VectorSubcoreMesh(core_axis_name="c", subcore_axis_name="s")

    @functools.partial(
        pl.kernel,
        out_type=jax.ShapeDtypeStruct((ep, d), F32),
        mesh=mesh,
        compiler_params=pltpu.CompilerParams(use_tc_tiling_on_sc=False),
        scratch_types=[
            pltpu.VMEM((g, SCCH), I32),
            pltpu.VMEM((g * SCCH, d), F32),
            pltpu.SemaphoreType.DMA,
        ],
    )
    def k(table_hbm, idx_hbm, out_hbm, idx_v, rows_v, sem):
        wid = lax.axis_index("s") * 2 + lax.axis_index("c")
        base_w = wid * per_w

        def body(j, carry):
            base = base_w + j * (SCCH * g)
            pltpu.sync_copy(idx_hbm.at[pl.ds(base // SCCH, g)], idx_v)
            _shift_idx2(idx_v, lvl, g)
            descs = [
                pltpu.async_copy(table_hbm.at[idx_v.at[r]],
                                 rows_v.at[pl.ds(r * SCCH, SCCH)], sem)
                for r in range(g)
            ]
            for dsc in descs:
                dsc.wait()
            pltpu.sync_copy(rows_v, out_hbm.at[pl.ds(base, SCCH * g)])
            return carry

        lax.fori_loop(0, ngrp, body, 0)

    return k(table, idx2)


def _sc_scatter(rows, idx2, lvl, np_l, d, ones_mode=False, col0=0, dfull=None):
    if dfull is None:
        dfull = d
    ep = idx2.shape[0] * SCCH
    per_w = ep // NW
    g = 2 if ones_mode else 1
    ngrp = per_w // (SCCH * g)
    rows_pt = np_l // 16
    zr = 8
    rbr = min(400, max(8, 4096 // d))
    while rows_pt % rbr:
        rbr //= 2
    mesh = plsc.VectorSubcoreMesh(core_axis_name="c", subcore_axis_name="s")

    scratch = [
        pltpu.VMEM((g, SCCH), I32),
        pltpu.VMEM((g * SCCH, d), F32),
        pltpu.VMEM((zr, d), F32),
        pltpu.VMEM((rbr, d), F32),
        pltpu.VMEM_SHARED((np_l, d), F32),
        pltpu.SemaphoreType.DMA,
    ]

    def body_common(rows_hbm, idx_hbm, out_hbm, idx_v, rows_v, zbuf, rbuf,
                    acc, sem):
        cid = lax.axis_index("c")
        sid = lax.axis_index("s")
        wid = sid * 2 + cid
        for r in range(zr):
            for t in range(d // 16):
                zbuf[r, pl.ds(t * 16, 16)] = jnp.zeros((16,), F32)
        r0 = sid * rows_pt

        def zb(j, carry):
            pltpu.sync_copy(zbuf, acc.at[pl.ds(r0 + j * zr, zr)])
            return carry

        lax.fori_loop(0, rows_pt // zr, zb, 0)

        if ones_mode:
            for r in range(g * SCCH):
                for t in range(d // 16):
                    rows_v[r, pl.ds(t * 16, 16)] = jnp.ones((16,), F32)

        plsc.subcore_barrier()
        base_w = wid * per_w

        def body(j, carry):
            base = base_w + j * (SCCH * g)
            pltpu.sync_copy(idx_hbm.at[pl.ds(base // SCCH, g)], idx_v)
            _shift_idx2(idx_v, lvl, g)
            if not ones_mode:
                if d == dfull:
                    pltpu.sync_copy(rows_hbm.at[pl.ds(base, SCCH * g)], rows_v)
                else:
                    pltpu.sync_copy(
                        rows_hbm.at[pl.ds(base, SCCH * g), pl.ds(col0, d)],
                        rows_v)
            for r in range(g):
                pltpu.sync_copy(rows_v.at[pl.ds(r * SCCH, SCCH)],
                                acc.at[idx_v.at[r]], add=True)
            return carry

        lax.fori_loop(0, ngrp, body, 0)
        plsc.subcore_barrier()

        def rb(j, carry):
            r = r0 + j * rbr
            pltpu.sync_copy(acc.at[pl.ds(r, rbr)], rbuf)
            pltpu.sync_copy(rbuf, out_hbm.at[cid].at[pl.ds(r, rbr)])
            return carry

        lax.fori_loop(0, rows_pt // rbr, rb, 0)

    if ones_mode:
        @functools.partial(
            pl.kernel,
            out_type=jax.ShapeDtypeStruct((2, np_l, d), F32),
            mesh=mesh, scratch_types=scratch,
            compiler_params=pltpu.CompilerParams(use_tc_tiling_on_sc=False),
        )
        def k1(idx_hbm, out_hbm, idx_v, rows_v, zbuf, rbuf, acc, sem):
            body_common(None, idx_hbm, out_hbm, idx_v, rows_v, zbuf, rbuf,
                        acc, sem)

        return k1(idx2)

    @functools.partial(
        pl.kernel,
        out_type=jax.ShapeDtypeStruct((2, np_l, d), F32),
        mesh=mesh, scratch_types=scratch,
        compiler_params=pltpu.CompilerParams(use_tc_tiling_on_sc=False),
    )
    def k2(rows_hbm, idx_hbm, out_hbm, idx_v, rows_v, zbuf, rbuf, acc, sem):
        body_common(rows_hbm, idx_hbm, out_hbm, idx_v, rows_v, zbuf, rbuf,
                    acc, sem)

    return k2(rows, idx2)


def _sc_scatter_add(rows, idx, lvl, np_l):
    d = rows.shape[1]
    if np_l * d * 4 > 5_000_000 and d > 128:
        parts = [_sc_scatter(rows, idx, lvl, np_l, 128, col0=c, dfull=d)
                 for c in range(0, d, 128)]
        return jnp.concatenate(parts, axis=2)
    return _sc_scatter(rows, idx, lvl, np_l, d)


def _sc_scatter_ones(idx, lvl, np_l):
    return _sc_scatter(None, idx, lvl, np_l, 16, ones_mode=True)


def kernel(pos, seq, ori, domain, seq_emb, params, x, edge_index, batch):
    np0 = LVL_NP[0]
    src = edge_index[0].astype(I32)
    dst = edge_index[1].astype(I32)
    src_p = jnp.concatenate([src, jnp.zeros((EP - E,), I32)])
    dst_p = jnp.concatenate([dst, jnp.full((EP - E,), N0, I32)])
    s2 = src_p.reshape(EP, 1)
    d2 = dst_p.reshape(EP, 1)
    src_p = src_p.reshape(EP // 128, 128)
    dst_p = dst_p.reshape(EP // 128, 128)

    g_tab = jnp.concatenate(
        [pos, jnp.zeros((N0, 1), F32), ori.reshape(N0, 9),
         jnp.zeros((N0, 3), F32)], axis=1)
    g_tab = jnp.pad(g_tab, ((0, np0 - N0), (0, 0)))

    x2 = jnp.pad(x.astype(I32), (0, np0 - N0),
                 constant_values=31).reshape(np0, 1)
    emb32 = jnp.pad(params["emb"], ((0, 11), (0, 0)))
    h = _embed_call(x2, emb32, np0)

    g0 = g1 = c0 = c1 = None
    for lvl in range(4):
        n_l = LVL_N[lvl]
        np_l = LVL_NP[lvl]
        iA, iB = 2 * lvl, 2 * lvl + 1
        k1a = jnp.concatenate([params[f"b{iA}_k1"], jnp.zeros((3, 24), F32)])
        k1b = jnp.concatenate([params[f"b{iB}_k1"], jnp.zeros((3, 24), F32)])
        k2a = params[f"b{iA}_k2"]
        k2b = params[f"b{iB}_k2"]

        gs = _sc_gather(g_tab, src_p, lvl)
        gd = _sc_gather(g_tab, dst_p, lvl)
        degp = _sc_scatter_ones(dst_p, lvl, np_l)

        hs_a = _sc_gather(h, src_p, lvl)
        msg_a, kern_b = _msg_a_call(gs, gd, s2, d2, hs_a,
                                    k1a, k2a, k1b, k2b, lvl)
        agg_a = _sc_scatter_add(msg_a, dst_p, lvl, np_l)
        h = _node_update(agg_a, degp, h, params[f"b{iA}_lin"],
                         params[f"b{iA}_res"], n_l, np_l)

        hs_b = _sc_gather(h, src_p, lvl)
        msg_b = _mult_call(hs_b, kern_b)
        agg_b = _sc_scatter_add(msg_b, dst_p, lvl, np_l)
        h = _node_update(agg_b, degp, h, params[f"b{iB}_lin"],
                         params[f"b{iB}_res"], n_l, np_l)

        if lvl < 3:
            m = np_l // 2
            g_tab = _pool_call(g_tab.reshape(m, 2, 16))
            h = _pool_call(h.reshape(m, 2, h.shape[1]))

    batch_l = batch[::8].astype(I32)
    epg = 8192
    batch_p = jnp.pad(batch_l, (0, epg - LVL_N[3]),
                      constant_values=B).reshape(epg // 128, 128)
    h_p = jnp.pad(h, ((0, epg - LVL_NP[3]), (0, 0)))
    gpart = _sc_scatter_add(h_p, batch_p, 0, 128)
    cpart = _sc_scatter_ones(batch_p, 0, 128)

    wc1 = params["Wc1"]
    out = _classifier_call(
        gpart[0, :B], gpart[1, :B], cpart[0, :B], cpart[1, :B],
        seq_emb, domain, params["Ws"], params["Wq"], params["Wd"],
        wc1[0:256], wc1[256:512], wc1[512:768], params["Wc2"])
    return out


# speedup vs baseline: 5.4971x; 1.0989x over previous
import functools

import jax
import jax.numpy as jnp
from jax import lax
from jax.experimental import pallas as pl
from jax.experimental.pallas import tpu as pltpu
from jax.experimental.pallas import tpu_sc as plsc

F32 = jnp.float32
I32 = jnp.int32

N0 = 50000
E = 800000
EP = 819200
EC = 3200
NEB = EP // EC
NCH = 1600
LVL_N = [50000, 25000, 12500, 6250]
LVL_NP = [51200, 25600, 12800, 6400]
B = 64
SEQ_L = 5.0
IO_CH = [(16, 32), (32, 32), (32, 64), (64, 64),
         (64, 128), (128, 128), (128, 256), (256, 256)]


def _embed_call(x2, emb32, np0):
    def body(x_ref, emb_ref, out_ref):
        lane = lax.broadcasted_iota(I32, (NCH, 32), 1)
        oh = (lane == x_ref[...]).astype(F32)
        out_ref[...] = jnp.dot(oh, emb_ref[...], preferred_element_type=F32)

    return pl.pallas_call(
        body,
        grid=(np0 // NCH,),
        in_specs=[
            pl.BlockSpec((NCH, 1), lambda i: (i, 0)),
            pl.BlockSpec((32, 16), lambda i: (0, 0)),
        ],
        out_specs=pl.BlockSpec((NCH, 16), lambda i: (i, 0)),
        out_shape=jax.ShapeDtypeStruct((np0, 16), F32),
    )(x2, emb32)


def _kern_call(gs, gd, s2, d2, k1a, k2a, k1b, k2b, lvl):
    ciA = k2a.shape[1]
    ciB = k2b.shape[1]
    scale = float(2 ** lvl) / SEQ_L

    def body(gs_ref, gd_ref, s_ref, d_ref, k1a_ref, k2a_ref,
             k1b_ref, k2b_ref, ka_ref, kb_ref):
        gsv = gs_ref[...]
        gdv = gd_ref[...]
        sl = jnp.right_shift(s_ref[...], lvl)
        dl = jnp.right_shift(d_ref[...], lvl)
        rel = (dl - sl).astype(F32) * scale
        lane = lax.broadcasted_iota(I32, (1, 16), 1)
        base = jnp.where(lane < 3, gdv - gsv, gsv * gdv)
        geo = base + rel * (lane == 3).astype(F32)
        ka = jnp.maximum(
            jnp.dot(geo, k1a_ref[...], preferred_element_type=F32), 0.0)
        ka_ref[...] = jnp.dot(ka, k2a_ref[...], preferred_element_type=F32)
        kb = jnp.maximum(
            jnp.dot(geo, k1b_ref[...], preferred_element_type=F32), 0.0)
        kb_ref[...] = jnp.dot(kb, k2b_ref[...], preferred_element_type=F32)

    return pl.pallas_call(
        body,
        grid=(NEB,),
        in_specs=[
            pl.BlockSpec((EC, 16), lambda i: (i, 0)),
            pl.BlockSpec((EC, 16), lambda i: (i, 0)),
            pl.BlockSpec((EC, 1), lambda i: (i, 0)),
            pl.BlockSpec((EC, 1), lambda i: (i, 0)),
            pl.BlockSpec((16, 24), lambda i: (0, 0)),
            pl.BlockSpec((24, ciA), lambda i: (0, 0)),
            pl.BlockSpec((16, 24), lambda i: (0, 0)),
            pl.BlockSpec((24, ciB), lambda i: (0, 0)),
        ],
        out_specs=[
            pl.BlockSpec((EC, ciA), lambda i: (i, 0)),
            pl.BlockSpec((EC, ciB), lambda i: (i, 0)),
        ],
        out_shape=[
            jax.ShapeDtypeStruct((EP, ciA), F32),
            jax.ShapeDtypeStruct((EP, ciB), F32),
        ],
    )(gs, gd, s2, d2, k1a, k2a, k1b, k2b)


def _mult_call(hs, kern):
    ci = hs.shape[1]

    def body(hs_ref, k_ref, out_ref):
        out_ref[...] = hs_ref[...] * k_ref[...]

    return pl.pallas_call(
        body,
        grid=(NEB,),
        in_specs=[
            pl.BlockSpec((EC, ci), lambda i: (i, 0)),
            pl.BlockSpec((EC, ci), lambda i: (i, 0)),
        ],
        out_specs=pl.BlockSpec((EC, ci), lambda i: (i, 0)),
        out_shape=jax.ShapeDtypeStruct((EP, ci), F32),
    )(hs, kern)


def _node_update(aggp, degp, h, lin, res, n_l, np_l):
    ci = lin.shape[0]
    co = lin.shape[1]
    nsteps = np_l // NCH
    inv_n = 1.0 / float(n_l)

    def body1(p0_ref, p1_ref, d0_ref, d1_ref, lin_ref, z_ref, st_ref):
        i = pl.program_id(0)
        deg = d0_ref[...][:, 0:1] + d1_ref[...][:, 0:1]
        agg = (p0_ref[...] + p1_ref[...]) / jnp.maximum(deg, 1.0)
        z = jnp.dot(agg, lin_ref[...], preferred_element_type=F32)
        z_ref[...] = z
        rid = i * NCH + lax.broadcasted_iota(I32, (NCH, 1), 0)
        m = (rid < n_l).astype(F32)
        zm = z * m
        s1 = jnp.sum(zm, axis=0, keepdims=True)
        s2 = jnp.sum(zm * z, axis=0, keepdims=True)

        @pl.when(i == 0)
        def _():
            st_ref[...] = jnp.zeros_like(st_ref)

        st_ref[0:1, :] += s1
        st_ref[1:2, :] += s2

    z, st = pl.pallas_call(
        body1,
        grid=(nsteps,),
        in_specs=[
            pl.BlockSpec((NCH, ci), lambda i: (i, 0)),
            pl.BlockSpec((NCH, ci), lambda i: (i, 0)),
            pl.BlockSpec((NCH, 16), lambda i: (i, 0)),
            pl.BlockSpec((NCH, 16), lambda i: (i, 0)),
            pl.BlockSpec((ci, co), lambda i: (0, 0)),
        ],
        out_specs=[
            pl.BlockSpec((NCH, co), lambda i: (i, 0)),
            pl.BlockSpec((8, co), lambda i: (0, 0)),
        ],
        out_shape=[
            jax.ShapeDtypeStruct((np_l, co), F32),
            jax.ShapeDtypeStruct((8, co), F32),
        ],
    )(aggp[0], aggp[1], degp[0], degp[1], lin)

    def body2(z_ref, st_ref, h_ref, res_ref, out_ref):
        mean = st_ref[0:1, :] * inv_n
        var = st_ref[1:2, :] * inv_n - mean * mean
        std = jnp.sqrt(jnp.maximum(var, 0.0))
        zn = (z_ref[...] - mean) / (std + 1e-5)
        out_ref[...] = jnp.maximum(zn, 0.0) + jnp.dot(
            h_ref[...], res_ref[...], preferred_element_type=F32)

    return pl.pallas_call(
        body2,
        grid=(nsteps,),
        in_specs=[
            pl.BlockSpec((NCH, co), lambda i: (i, 0)),
            pl.BlockSpec((8, co), lambda i: (0, 0)),
            pl.BlockSpec((NCH, ci), lambda i: (i, 0)),
            pl.BlockSpec((ci, co), lambda i: (0, 0)),
        ],
        out_specs=pl.BlockSpec((NCH, co), lambda i: (i, 0)),
        out_shape=jax.ShapeDtypeStruct((np_l, co), F32),
    )(z, st, h, res)


def _pool_call(a3):
    m = a3.shape[0]
    d = a3.shape[2]

    def body(a_ref, out_ref):
        out_ref[...] = (a_ref[:, 0, :] + a_ref[:, 1, :]) * 0.5

    return pl.pallas_call(
        body,
        grid=(m // NCH,),
        in_specs=[pl.BlockSpec((NCH, 2, d), lambda i: (i, 0, 0))],
        out_specs=pl.BlockSpec((NCH, d), lambda i: (i, 0)),
        out_shape=jax.ShapeDtypeStruct((m, d), F32),
    )(a3)


def _classifier_call(g0, g1, c0, c1, seq_emb, domain, ws, wq, wd,
                     wc1a, wc1b, wc1c, wc2):
    def body(g0_ref, g1_ref, c0_ref, c1_ref, se_ref, dom_ref, ws_ref,
             wq_ref, wd_ref, a_ref, b_ref, c_ref, w2_ref, out_ref):
        cnt = c0_ref[...][:, 0:1] + c1_ref[...][:, 0:1]
        g = (g0_ref[...] + g1_ref[...]) / jnp.maximum(cnt, 1.0)
        struct = jnp.dot(g, ws_ref[...], preferred_element_type=F32)
        seqf = jnp.dot(se_ref[...], wq_ref[...], preferred_element_type=F32)
        dom = dom_ref[...]
        mask = jnp.sum(dom, axis=1, keepdims=True) != 0.0
        domf = jnp.where(mask,
                         jnp.dot(dom, wd_ref[...], preferred_element_type=F32),
                         0.0)
        hid = (jnp.dot(struct, a_ref[...], preferred_element_type=F32)
               + jnp.dot(seqf, b_ref[...], preferred_element_type=F32)
               + jnp.dot(domf, c_ref[...], preferred_element_type=F32))
        mean = jnp.mean(hid, axis=0, keepdims=True)
        var = jnp.mean(hid * hid, axis=0, keepdims=True) - mean * mean
        std = jnp.sqrt(jnp.maximum(var, 0.0))
        hid = jnp.maximum((hid - mean) / (std + 1e-5), 0.0)
        out_ref[...] = jnp.dot(hid, w2_ref[...], preferred_element_type=F32)

    nc = wc2.shape[1]
    return pl.pallas_call(
        body,
        out_shape=jax.ShapeDtypeStruct((B, nc), F32),
    )(g0, g1, c0, c1, seq_emb, domain, ws, wq, wd, wc1a, wc1b, wc1c, wc2)


NW = 32
SCCH = 128


def _shift_idx2(idx_v, lvl, g):
    if lvl:
        for r in range(g):
            for t in range(SCCH // 16):
                sl = pl.ds(t * 16, 16)
                idx_v[r, sl] = jnp.right_shift(idx_v[r, sl], lvl)


def _grp(d):
    return max(1, min(8, 262144 // (SCCH * d * 4)))


def _sc_gather(table, idx2, lvl):
    d = table.shape[1]
    ep = idx2.shape[0] * SCCH
    per_w = ep // NW
    g = _grp(d)
    ngrp = per_w // (SCCH * g)
    mesh = plsc.VectorSubcoreMesh(core_axis_name="c", subcore_axis_name="s")

    @functools.partial(
        pl.kernel,
        out_type=jax.ShapeDtypeStruct((ep, d), F32),
        mesh=mesh,
        compiler_params=pltpu.CompilerParams(use_tc_tiling_on_sc=False),
        scratch_types=[
            pltpu.VMEM((g, SCCH), I32),
            pltpu.VMEM((g * SCCH, d), F32),
            pltpu.SemaphoreType.DMA,
        ],
    )
    def k(table_hbm, idx_hbm, out_hbm, idx_v, rows_v, sem):
        wid = lax.axis_index("s") * 2 + lax.axis_index("c")
        base_w = wid * per_w

        def body(j, carry):
            base = base_w + j * (SCCH * g)
            pltpu.sync_copy(idx_hbm.at[pl.ds(base // SCCH, g)], idx_v)
            _shift_idx2(idx_v, lvl, g)
            descs = [
                pltpu.async_copy(table_hbm.at[idx_v.at[r]],
                                 rows_v.at[pl.ds(r * SCCH, SCCH)], sem)
                for r in range(g)
            ]
            for dsc in descs:
                dsc.wait()
            pltpu.sync_copy(rows_v, out_hbm.at[pl.ds(base, SCCH * g)])
            return carry

        lax.fori_loop(0, ngrp, body, 0)

    return k(table, idx2)


def _sc_scatter(rows, idx2, lvl, np_l, d, ones_mode=False, col0=0, dfull=None):
    if dfull is None:
        dfull = d
    ep = idx2.shape[0] * SCCH
    per_w = ep // NW
    g = 2 if ones_mode else 1
    ngrp = per_w // (SCCH * g)
    rows_pt = np_l // 16
    zr = 8
    rbr = min(400, max(8, 4096 // d))
    while rows_pt % rbr:
        rbr //= 2
    mesh = plsc.VectorSubcoreMesh(core_axis_name="c", subcore_axis_name="s")

    scratch = [
        pltpu.VMEM((g, SCCH), I32),
        pltpu.VMEM((g * SCCH, d), F32),
        pltpu.VMEM((zr, d), F32),
        pltpu.VMEM((rbr, d), F32),
        pltpu.VMEM_SHARED((np_l, d), F32),
        pltpu.SemaphoreType.DMA,
    ]

    def body_common(rows_hbm, idx_hbm, out_hbm, idx_v, rows_v, zbuf, rbuf,
                    acc, sem):
        cid = lax.axis_index("c")
        sid = lax.axis_index("s")
        wid = sid * 2 + cid
        for r in range(zr):
            for t in range(d // 16):
                zbuf[r, pl.ds(t * 16, 16)] = jnp.zeros((16,), F32)
        r0 = sid * rows_pt

        def zb(j, carry):
            pltpu.sync_copy(zbuf, acc.at[pl.ds(r0 + j * zr, zr)])
            return carry

        lax.fori_loop(0, rows_pt // zr, zb, 0)

        if ones_mode:
            for r in range(g * SCCH):
                for t in range(d // 16):
                    rows_v[r, pl.ds(t * 16, 16)] = jnp.ones((16,), F32)

        plsc.subcore_barrier()
        base_w = wid * per_w

        def body(j, carry):
            base = base_w + j * (SCCH * g)
            pltpu.sync_copy(idx_hbm.at[pl.ds(base // SCCH, g)], idx_v)
            _shift_idx2(idx_v, lvl, g)
            if not ones_mode:
                if d == dfull:
                    pltpu.sync_copy(rows_hbm.at[pl.ds(base, SCCH * g)], rows_v)
                else:
                    pltpu.sync_copy(
                        rows_hbm.at[pl.ds(base, SCCH * g), pl.ds(col0, d)],
                        rows_v)
            for r in range(g):
                pltpu.sync_copy(rows_v.at[pl.ds(r * SCCH, SCCH)],
                                acc.at[idx_v.at[r]], add=True)
            return carry

        lax.fori_loop(0, ngrp, body, 0)
        plsc.subcore_barrier()

        def rb(j, carry):
            r = r0 + j * rbr
            pltpu.sync_copy(acc.at[pl.ds(r, rbr)], rbuf)
            pltpu.sync_copy(rbuf, out_hbm.at[cid].at[pl.ds(r, rbr)])
            return carry

        lax.fori_loop(0, rows_pt // rbr, rb, 0)

    if ones_mode:
        @functools.partial(
            pl.kernel,
            out_type=jax.ShapeDtypeStruct((2, np_l, d), F32),
            mesh=mesh, scratch_types=scratch,
            compiler_params=pltpu.CompilerParams(use_tc_tiling_on_sc=False),
        )
        def k1(idx_hbm, out_hbm, idx_v, rows_v, zbuf, rbuf, acc, sem):
            body_common(None, idx_hbm, out_hbm, idx_v, rows_v, zbuf, rbuf,
                        acc, sem)

        return k1(idx2)

    @functools.partial(
        pl.kernel,
        out_type=jax.ShapeDtypeStruct((2, np_l, d), F32),
        mesh=mesh, scratch_types=scratch,
        compiler_params=pltpu.CompilerParams(use_tc_tiling_on_sc=False),
    )
    def k2(rows_hbm, idx_hbm, out_hbm, idx_v, rows_v, zbuf, rbuf, acc, sem):
        body_common(rows_hbm, idx_hbm, out_hbm, idx_v, rows_v, zbuf, rbuf,
                    acc, sem)

    return k2(rows, idx2)


def _sc_gms(h_tab, kern, src2, dst2, lvl, np_l, col0=0):
    d = h_tab.shape[1]
    dk = kern.shape[1]
    ep = src2.shape[0] * SCCH
    per_w = ep // NW
    nch = per_w // SCCH
    rows_pt = np_l // 16
    zr = 8
    rbr = min(400, max(8, 4096 // d))
    while rows_pt % rbr:
        rbr //= 2
    mesh = plsc.VectorSubcoreMesh(core_axis_name="c", subcore_axis_name="s")

    @functools.partial(
        pl.kernel,
        out_type=jax.ShapeDtypeStruct((2, np_l, d), F32),
        mesh=mesh,
        compiler_params=pltpu.CompilerParams(use_tc_tiling_on_sc=False),
        scratch_types=[
            pltpu.VMEM((1, SCCH), I32),
            pltpu.VMEM((1, SCCH), I32),
            pltpu.VMEM((SCCH, d), F32),
            pltpu.VMEM((SCCH, d), F32),
            pltpu.VMEM((zr, d), F32),
            pltpu.VMEM((rbr, d), F32),
            pltpu.VMEM_SHARED((np_l, d), F32),
            pltpu.SemaphoreType.DMA,
        ],
    )
    def k(h_hbm, kern_hbm, src_hbm, dst_hbm, out_hbm, siv, div, hv, kv,
          zbuf, rbuf, acc, sem):
        cid = lax.axis_index("c")
        sid = lax.axis_index("s")
        wid = sid * 2 + cid
        for r in range(zr):
            for t in range(d // 16):
                zbuf[r, pl.ds(t * 16, 16)] = jnp.zeros((16,), F32)
        r0 = sid * rows_pt

        def zb(j, carry):
            pltpu.sync_copy(zbuf, acc.at[pl.ds(r0 + j * zr, zr)])
            return carry

        lax.fori_loop(0, rows_pt // zr, zb, 0)
        plsc.subcore_barrier()
        base_w = wid * per_w

        def body(j, carry):
            base = base_w + j * SCCH
            row = base // SCCH
            pltpu.sync_copy(src_hbm.at[pl.ds(row, 1)], siv)
            pltpu.sync_copy(dst_hbm.at[pl.ds(row, 1)], div)
            _shift_idx2(siv, lvl, 1)
            _shift_idx2(div, lvl, 1)
            dsc = pltpu.async_copy(h_hbm.at[siv.at[0]], hv, sem)
            if col0 == 0 and d == dk:
                pltpu.sync_copy(kern_hbm.at[pl.ds(base, SCCH)], kv)
            else:
                pltpu.sync_copy(
                    kern_hbm.at[pl.ds(base, SCCH), pl.ds(col0, d)], kv)
            dsc.wait()
            for r in range(SCCH):
                for t in range(d // 16):
                    sl = pl.ds(t * 16, 16)
                    kv[r, sl] = kv[r, sl] * hv[r, sl]
            pltpu.sync_copy(kv, acc.at[div.at[0]], add=True)
            return carry

        lax.fori_loop(0, nch, body, 0)
        plsc.subcore_barrier()

        def rb(j, carry):
            r = r0 + j * rbr
            pltpu.sync_copy(acc.at[pl.ds(r, rbr)], rbuf)
            pltpu.sync_copy(rbuf, out_hbm.at[cid].at[pl.ds(r, rbr)])
            return carry

        lax.fori_loop(0, rows_pt // rbr, rb, 0)

    return k(h_tab, kern, src2, dst2)


def _gms_dispatch(h, kern, src2, dst2, lvl, np_l):
    d = kern.shape[1]
    if np_l * d * 4 > 5_000_000:
        dh = d // 2
        parts = [_sc_gms(h[:, c:c + dh], kern, src2, dst2, lvl, np_l,
                         col0=c) for c in range(0, d, dh)]
        return jnp.concatenate(parts, axis=2)
    return _sc_gms(h, kern, src2, dst2, lvl, np_l)


def _sc_scatter_add(rows, idx, lvl, np_l):
    d = rows.shape[1]
    if np_l * d * 4 > 5_000_000 and d > 128:
        parts = [_sc_scatter(rows, idx, lvl, np_l, 128, col0=c, dfull=d)
                 for c in range(0, d, 128)]
        return jnp.concatenate(parts, axis=2)
    return _sc_scatter(rows, idx, lvl, np_l, d)


def _sc_scatter_ones(idx, lvl, np_l):
    return _sc_scatter(None, idx, lvl, np_l, 16, ones_mode=True)


def kernel(pos, seq, ori, domain, seq_emb, params, x, edge_index, batch):
    np0 = LVL_NP[0]
    src = edge_index[0].astype(I32)
    dst = edge_index[1].astype(I32)
    src_p = jnp.concatenate([src, jnp.zeros((EP - E,), I32)])
    dst_p = jnp.concatenate([dst, jnp.full((EP - E,), N0, I32)])
    s2 = src_p.reshape(EP, 1)
    d2 = dst_p.reshape(EP, 1)
    src_p = src_p.reshape(EP // 128, 128)
    dst_p = dst_p.reshape(EP // 128, 128)

    g_tab = jnp.concatenate(
        [pos, jnp.zeros((N0, 1), F32), ori.reshape(N0, 9),
         jnp.zeros((N0, 3), F32)], axis=1)
    g_tab = jnp.pad(g_tab, ((0, np0 - N0), (0, 0)))

    x2 = jnp.pad(x.astype(I32), (0, np0 - N0),
                 constant_values=31).reshape(np0, 1)
    emb32 = jnp.pad(params["emb"], ((0, 11), (0, 0)))
    h = _embed_call(x2, emb32, np0)

    g0 = g1 = c0 = c1 = None
    for lvl in range(4):
        n_l = LVL_N[lvl]
        np_l = LVL_NP[lvl]
        iA, iB = 2 * lvl, 2 * lvl + 1
        k1a = jnp.concatenate([params[f"b{iA}_k1"], jnp.zeros((3, 24), F32)])
        k1b = jnp.concatenate([params[f"b{iB}_k1"], jnp.zeros((3, 24), F32)])
        k2a = params[f"b{iA}_k2"]
        k2b = params[f"b{iB}_k2"]

        gs = _sc_gather(g_tab, src_p, lvl)
        gd = _sc_gather(g_tab, dst_p, lvl)
        degp = _sc_scatter_ones(dst_p, lvl, np_l)

        kern_a, kern_b = _kern_call(gs, gd, s2, d2,
                                    k1a, k2a, k1b, k2b, lvl)
        agg_a = _gms_dispatch(h, kern_a, src_p, dst_p, lvl, np_l)
        h = _node_update(agg_a, degp, h, params[f"b{iA}_lin"],
                         params[f"b{iA}_res"], n_l, np_l)

        agg_b = _gms_dispatch(h, kern_b, src_p, dst_p, lvl, np_l)
        h = _node_update(agg_b, degp, h, params[f"b{iB}_lin"],
                         params[f"b{iB}_res"], n_l, np_l)

        if lvl < 3:
            m = np_l // 2
            g_tab = _pool_call(g_tab.reshape(m, 2, 16))
            h = _pool_call(h.reshape(m, 2, h.shape[1]))

    batch_l = batch[::8].astype(I32)
    epg = 8192
    batch_p = jnp.pad(batch_l, (0, epg - LVL_N[3]),
                      constant_values=B).reshape(epg // 128, 128)
    h_p = jnp.pad(h, ((0, epg - LVL_NP[3]), (0, 0)))
    gpart = _sc_scatter_add(h_p, batch_p, 0, 128)
    cpart = _sc_scatter_ones(batch_p, 0, 128)

    wc1 = params["Wc1"]
    out = _classifier_call(
        gpart[0, :B], gpart[1, :B], cpart[0, :B], cpart[1, :B],
        seq_emb, domain, params["Ws"], params["Wq"], params["Wd"],
        wc1[0:256], wc1[256:512], wc1[512:768], params["Wc2"])
    return out


# speedup vs baseline: 7.0781x; 1.2876x over previous
import functools

import jax
import jax.numpy as jnp
from jax import lax
from jax.experimental import pallas as pl
from jax.experimental.pallas import tpu as pltpu
from jax.experimental.pallas import tpu_sc as plsc

F32 = jnp.float32
I32 = jnp.int32

N0 = 50000
E = 800000
EP = 819200
EC = 3200
NEB = EP // EC
NCH = 1600
LVL_N = [50000, 25000, 12500, 6250]
LVL_NP = [51200, 25600, 12800, 6400]
B = 64
SEQ_L = 5.0
IO_CH = [(16, 32), (32, 32), (32, 64), (64, 64),
         (64, 128), (128, 128), (128, 256), (256, 256)]


def _embed_call(x2, emb32, np0):
    def body(x_ref, emb_ref, out_ref):
        lane = lax.broadcasted_iota(I32, (NCH, 32), 1)
        oh = (lane == x_ref[...]).astype(F32)
        out_ref[...] = jnp.dot(oh, emb_ref[...], preferred_element_type=F32)

    return pl.pallas_call(
        body,
        grid=(np0 // NCH,),
        in_specs=[
            pl.BlockSpec((NCH, 1), lambda i: (i, 0)),
            pl.BlockSpec((32, 16), lambda i: (0, 0)),
        ],
        out_specs=pl.BlockSpec((NCH, 16), lambda i: (i, 0)),
        out_shape=jax.ShapeDtypeStruct((np0, 16), F32),
    )(x2, emb32)


def _kern_call(gs, gd, s2, d2, k1a, k2a, k1b, k2b, lvl):
    ciA = k2a.shape[1]
    ciB = k2b.shape[1]
    scale = float(2 ** lvl) / SEQ_L

    def body(gs_ref, gd_ref, s_ref, d_ref, k1a_ref, k2a_ref,
             k1b_ref, k2b_ref, ka_ref, kb_ref):
        gsv = gs_ref[...]
        gdv = gd_ref[...]
        sl = jnp.right_shift(s_ref[...], lvl)
        dl = jnp.right_shift(d_ref[...], lvl)
        rel = (dl - sl).astype(F32) * scale
        lane = lax.broadcasted_iota(I32, (1, 16), 1)
        base = jnp.where(lane < 3, gdv - gsv, gsv * gdv)
        geo = base + rel * (lane == 3).astype(F32)
        ka = jnp.maximum(
            jnp.dot(geo, k1a_ref[...], preferred_element_type=F32), 0.0)
        ka_ref[...] = jnp.dot(ka, k2a_ref[...], preferred_element_type=F32)
        kb = jnp.maximum(
            jnp.dot(geo, k1b_ref[...], preferred_element_type=F32), 0.0)
        kb_ref[...] = jnp.dot(kb, k2b_ref[...], preferred_element_type=F32)

    return pl.pallas_call(
        body,
        grid=(NEB,),
        in_specs=[
            pl.BlockSpec((EC, 16), lambda i: (i, 0)),
            pl.BlockSpec((EC, 16), lambda i: (i, 0)),
            pl.BlockSpec((EC, 1), lambda i: (i, 0)),
            pl.BlockSpec((EC, 1), lambda i: (i, 0)),
            pl.BlockSpec((16, 24), lambda i: (0, 0)),
            pl.BlockSpec((24, ciA), lambda i: (0, 0)),
            pl.BlockSpec((16, 24), lambda i: (0, 0)),
            pl.BlockSpec((24, ciB), lambda i: (0, 0)),
        ],
        out_specs=[
            pl.BlockSpec((EC, ciA), lambda i: (i, 0)),
            pl.BlockSpec((EC, ciB), lambda i: (i, 0)),
        ],
        out_shape=[
            jax.ShapeDtypeStruct((EP, ciA), F32),
            jax.ShapeDtypeStruct((EP, ciB), F32),
        ],
    )(gs, gd, s2, d2, k1a, k2a, k1b, k2b)


def _mult_call(hs, kern):
    ci = hs.shape[1]

    def body(hs_ref, k_ref, out_ref):
        out_ref[...] = hs_ref[...] * k_ref[...]

    return pl.pallas_call(
        body,
        grid=(NEB,),
        in_specs=[
            pl.BlockSpec((EC, ci), lambda i: (i, 0)),
            pl.BlockSpec((EC, ci), lambda i: (i, 0)),
        ],
        out_specs=pl.BlockSpec((EC, ci), lambda i: (i, 0)),
        out_shape=jax.ShapeDtypeStruct((EP, ci), F32),
    )(hs, kern)


def _node_update(aggp, degp, h, lin, res, n_l, np_l):
    ci = lin.shape[0]
    co = lin.shape[1]
    nsteps = np_l // NCH
    inv_n = 1.0 / float(n_l)

    def body1(p0_ref, p1_ref, d0_ref, d1_ref, lin_ref, z_ref, st_ref):
        i = pl.program_id(0)
        deg = d0_ref[...][:, 0:1] + d1_ref[...][:, 0:1]
        agg = (p0_ref[...] + p1_ref[...]) / jnp.maximum(deg, 1.0)
        z = jnp.dot(agg, lin_ref[...], preferred_element_type=F32)
        z_ref[...] = z
        rid = i * NCH + lax.broadcasted_iota(I32, (NCH, 1), 0)
        m = (rid < n_l).astype(F32)
        zm = z * m
        s1 = jnp.sum(zm, axis=0, keepdims=True)
        s2 = jnp.sum(zm * z, axis=0, keepdims=True)

        @pl.when(i == 0)
        def _():
            st_ref[...] = jnp.zeros_like(st_ref)

        st_ref[0:1, :] += s1
        st_ref[1:2, :] += s2

    z, st = pl.pallas_call(
        body1,
        grid=(nsteps,),
        in_specs=[
            pl.BlockSpec((NCH, ci), lambda i: (i, 0)),
            pl.BlockSpec((NCH, ci), lambda i: (i, 0)),
            pl.BlockSpec((NCH, 16), lambda i: (i, 0)),
            pl.BlockSpec((NCH, 16), lambda i: (i, 0)),
            pl.BlockSpec((ci, co), lambda i: (0, 0)),
        ],
        out_specs=[
            pl.BlockSpec((NCH, co), lambda i: (i, 0)),
            pl.BlockSpec((8, co), lambda i: (0, 0)),
        ],
        out_shape=[
            jax.ShapeDtypeStruct((np_l, co), F32),
            jax.ShapeDtypeStruct((8, co), F32),
        ],
    )(aggp[0], aggp[1], degp[0], degp[1], lin)

    def body2(z_ref, st_ref, h_ref, res_ref, out_ref):
        mean = st_ref[0:1, :] * inv_n
        var = st_ref[1:2, :] * inv_n - mean * mean
        std = jnp.sqrt(jnp.maximum(var, 0.0))
        zn = (z_ref[...] - mean) / (std + 1e-5)
        out_ref[...] = jnp.maximum(zn, 0.0) + jnp.dot(
            h_ref[...], res_ref[...], preferred_element_type=F32)

    return pl.pallas_call(
        body2,
        grid=(nsteps,),
        in_specs=[
            pl.BlockSpec((NCH, co), lambda i: (i, 0)),
            pl.BlockSpec((8, co), lambda i: (0, 0)),
            pl.BlockSpec((NCH, ci), lambda i: (i, 0)),
            pl.BlockSpec((ci, co), lambda i: (0, 0)),
        ],
        out_specs=pl.BlockSpec((NCH, co), lambda i: (i, 0)),
        out_shape=jax.ShapeDtypeStruct((np_l, co), F32),
    )(z, st, h, res)


def _pool_call(a3):
    m = a3.shape[0]
    d = a3.shape[2]

    def body(a_ref, out_ref):
        out_ref[...] = (a_ref[:, 0, :] + a_ref[:, 1, :]) * 0.5

    return pl.pallas_call(
        body,
        grid=(m // NCH,),
        in_specs=[pl.BlockSpec((NCH, 2, d), lambda i: (i, 0, 0))],
        out_specs=pl.BlockSpec((NCH, d), lambda i: (i, 0)),
        out_shape=jax.ShapeDtypeStruct((m, d), F32),
    )(a3)


def _classifier_call(g0, g1, c0, c1, seq_emb, domain, ws, wq, wd,
                     wc1a, wc1b, wc1c, wc2):
    def body(g0_ref, g1_ref, c0_ref, c1_ref, se_ref, dom_ref, ws_ref,
             wq_ref, wd_ref, a_ref, b_ref, c_ref, w2_ref, out_ref):
        cnt = c0_ref[...][:, 0:1] + c1_ref[...][:, 0:1]
        g = (g0_ref[...] + g1_ref[...]) / jnp.maximum(cnt, 1.0)
        struct = jnp.dot(g, ws_ref[...], preferred_element_type=F32)
        seqf = jnp.dot(se_ref[...], wq_ref[...], preferred_element_type=F32)
        dom = dom_ref[...]
        mask = jnp.sum(dom, axis=1, keepdims=True) != 0.0
        domf = jnp.where(mask,
                         jnp.dot(dom, wd_ref[...], preferred_element_type=F32),
                         0.0)
        hid = (jnp.dot(struct, a_ref[...], preferred_element_type=F32)
               + jnp.dot(seqf, b_ref[...], preferred_element_type=F32)
               + jnp.dot(domf, c_ref[...], preferred_element_type=F32))
        mean = jnp.mean(hid, axis=0, keepdims=True)
        var = jnp.mean(hid * hid, axis=0, keepdims=True) - mean * mean
        std = jnp.sqrt(jnp.maximum(var, 0.0))
        hid = jnp.maximum((hid - mean) / (std + 1e-5), 0.0)
        out_ref[...] = jnp.dot(hid, w2_ref[...], preferred_element_type=F32)

    nc = wc2.shape[1]
    return pl.pallas_call(
        body,
        out_shape=jax.ShapeDtypeStruct((B, nc), F32),
    )(g0, g1, c0, c1, seq_emb, domain, ws, wq, wd, wc1a, wc1b, wc1c, wc2)


NW = 32
SCCH = 128


def _shift_idx2(idx_v, lvl, g):
    if lvl:
        for r in range(g):
            for t in range(SCCH // 16):
                sl = pl.ds(t * 16, 16)
                idx_v[r, sl] = jnp.right_shift(idx_v[r, sl], lvl)


def _grp(d):
    return max(1, min(8, 262144 // (SCCH * d * 4)))


def _sc_gather(table, idx2, lvl):
    d = table.shape[1]
    ep = idx2.shape[0] * SCCH
    per_w = ep // NW
    g = _grp(d)
    ngrp = per_w // (SCCH * g)
    mesh = plsc.VectorSubcoreMesh(core_axis_name="c", subcore_axis_name="s")

    @functools.partial(
        pl.kernel,
        out_type=jax.ShapeDtypeStruct((ep, d), F32),
        mesh=mesh,
        compiler_params=pltpu.CompilerParams(use_tc_tiling_on_sc=False),
        scratch_types=[
            pltpu.VMEM((g, SCCH), I32),
            pltpu.VMEM((g * SCCH, d), F32),
            pltpu.SemaphoreType.DMA,
        ],
    )
    def k(table_hbm, idx_hbm, out_hbm, idx_v, rows_v, sem):
        wid = lax.axis_index("s") * 2 + lax.axis_index("c")
        base_w = wid * per_w

        def body(j, carry):
            base = base_w + j * (SCCH * g)
            pltpu.sync_copy(idx_hbm.at[pl.ds(base // SCCH, g)], idx_v)
            _shift_idx2(idx_v, lvl, g)
            descs = [
                pltpu.async_copy(table_hbm.at[idx_v.at[r]],
                                 rows_v.at[pl.ds(r * SCCH, SCCH)], sem)
                for r in range(g)
            ]
            for dsc in descs:
                dsc.wait()
            pltpu.sync_copy(rows_v, out_hbm.at[pl.ds(base, SCCH * g)])
            return carry

        lax.fori_loop(0, ngrp, body, 0)

    return k(table, idx2)


def _sc_scatter(rows, idx2, lvl, np_l, d, ones_mode=False, col0=0, dfull=None):
    if dfull is None:
        dfull = d
    ep = idx2.shape[0] * SCCH
    per_w = ep // NW
    g = 2 if ones_mode else 1
    ngrp = per_w // (SCCH * g)
    rows_pt = np_l // 16
    zr = 8
    rbr = min(400, max(8, 4096 // d))
    while rows_pt % rbr:
        rbr //= 2
    mesh = plsc.VectorSubcoreMesh(core_axis_name="c", subcore_axis_name="s")

    scratch = [
        pltpu.VMEM((g, SCCH), I32),
        pltpu.VMEM((g * SCCH, d), F32),
        pltpu.VMEM((zr, d), F32),
        pltpu.VMEM((rbr, d), F32),
        pltpu.VMEM_SHARED((np_l, d), F32),
        pltpu.SemaphoreType.DMA,
    ]

    def body_common(rows_hbm, idx_hbm, out_hbm, idx_v, rows_v, zbuf, rbuf,
                    acc, sem):
        cid = lax.axis_index("c")
        sid = lax.axis_index("s")
        wid = sid * 2 + cid
        for r in range(zr):
            for t in range(d // 16):
                zbuf[r, pl.ds(t * 16, 16)] = jnp.zeros((16,), F32)
        r0 = sid * rows_pt

        def zb(j, carry):
            pltpu.sync_copy(zbuf, acc.at[pl.ds(r0 + j * zr, zr)])
            return carry

        lax.fori_loop(0, rows_pt // zr, zb, 0)

        if ones_mode:
            for r in range(g * SCCH):
                for t in range(d // 16):
                    rows_v[r, pl.ds(t * 16, 16)] = jnp.ones((16,), F32)

        plsc.subcore_barrier()
        base_w = wid * per_w

        def body(j, carry):
            base = base_w + j * (SCCH * g)
            pltpu.sync_copy(idx_hbm.at[pl.ds(base // SCCH, g)], idx_v)
            _shift_idx2(idx_v, lvl, g)
            if not ones_mode:
                if d == dfull:
                    pltpu.sync_copy(rows_hbm.at[pl.ds(base, SCCH * g)], rows_v)
                else:
                    pltpu.sync_copy(
                        rows_hbm.at[pl.ds(base, SCCH * g), pl.ds(col0, d)],
                        rows_v)
            for r in range(g):
                pltpu.sync_copy(rows_v.at[pl.ds(r * SCCH, SCCH)],
                                acc.at[idx_v.at[r]], add=True)
            return carry

        lax.fori_loop(0, ngrp, body, 0)
        plsc.subcore_barrier()

        def rb(j, carry):
            r = r0 + j * rbr
            pltpu.sync_copy(acc.at[pl.ds(r, rbr)], rbuf)
            pltpu.sync_copy(rbuf, out_hbm.at[cid].at[pl.ds(r, rbr)])
            return carry

        lax.fori_loop(0, rows_pt // rbr, rb, 0)

    if ones_mode:
        @functools.partial(
            pl.kernel,
            out_type=jax.ShapeDtypeStruct((2, np_l, d), F32),
            mesh=mesh, scratch_types=scratch,
            compiler_params=pltpu.CompilerParams(use_tc_tiling_on_sc=False),
        )
        def k1(idx_hbm, out_hbm, idx_v, rows_v, zbuf, rbuf, acc, sem):
            body_common(None, idx_hbm, out_hbm, idx_v, rows_v, zbuf, rbuf,
                        acc, sem)

        return k1(idx2)

    @functools.partial(
        pl.kernel,
        out_type=jax.ShapeDtypeStruct((2, np_l, d), F32),
        mesh=mesh, scratch_types=scratch,
        compiler_params=pltpu.CompilerParams(use_tc_tiling_on_sc=False),
    )
    def k2(rows_hbm, idx_hbm, out_hbm, idx_v, rows_v, zbuf, rbuf, acc, sem):
        body_common(rows_hbm, idx_hbm, out_hbm, idx_v, rows_v, zbuf, rbuf,
                    acc, sem)

    return k2(rows, idx2)


def _sc_gms(h_tab, kern, src2, dst2, lvl, np_l, col0=0):
    d = h_tab.shape[1]
    dk = kern.shape[1]
    ep = src2.shape[0] * SCCH
    per_w = ep // NW
    nch = per_w // SCCH
    rows_pt = np_l // 16
    zr = 8
    rbr = min(400, max(8, 4096 // d))
    while rows_pt % rbr:
        rbr //= 2
    mesh = plsc.VectorSubcoreMesh(core_axis_name="c", subcore_axis_name="s")

    @functools.partial(
        pl.kernel,
        out_type=jax.ShapeDtypeStruct((2, np_l, d), F32),
        mesh=mesh,
        compiler_params=pltpu.CompilerParams(use_tc_tiling_on_sc=False),
        scratch_types=[
            pltpu.VMEM((2, SCCH), I32),
            pltpu.VMEM((2, SCCH), I32),
            pltpu.VMEM((2 * SCCH, d), F32),
            pltpu.VMEM((2 * SCCH, d), F32),
            pltpu.VMEM((zr, d), F32),
            pltpu.VMEM((rbr, d), F32),
            pltpu.VMEM_SHARED((np_l, d), F32),
            pltpu.SemaphoreType.DMA,
            pltpu.SemaphoreType.DMA,
        ],
    )
    def k(h_hbm, kern_hbm, src_hbm, dst_hbm, out_hbm, siv, div, hv, kv,
          zbuf, rbuf, acc, sem_i, sem_g):
        cid = lax.axis_index("c")
        sid = lax.axis_index("s")
        wid = sid * 2 + cid
        for r in range(zr):
            for t in range(d // 16):
                zbuf[r, pl.ds(t * 16, 16)] = jnp.zeros((16,), F32)
        r0 = sid * rows_pt

        def zb(j, carry):
            pltpu.sync_copy(zbuf, acc.at[pl.ds(r0 + j * zr, zr)])
            return carry

        lax.fori_loop(0, rows_pt // zr, zb, 0)
        plsc.subcore_barrier()
        base_w = wid * per_w
        row_w = base_w // SCCH

        def shift_row(ref, b):
            if lvl:
                for t in range(SCCH // 16):
                    sl = pl.ds(t * 16, 16)
                    ref[b, sl] = jnp.right_shift(ref[b, sl], lvl)

        def kern_src(j):
            if col0 == 0 and d == dk:
                return kern_hbm.at[pl.ds(base_w + j * SCCH, SCCH)]
            return kern_hbm.at[pl.ds(base_w + j * SCCH, SCCH),
                               pl.ds(col0, d)]

        def start_group(j, b):
            shift_row(siv, b)
            shift_row(div, b)
            pltpu.async_copy(h_hbm.at[siv.at[b]],
                             hv.at[pl.ds(b * SCCH, SCCH)], sem_g)
            pltpu.async_copy(kern_src(j), kv.at[pl.ds(b * SCCH, SCCH)],
                             sem_g)

        def wait_group(b):
            pltpu.make_async_copy(
                h_hbm.at[siv.at[b]], hv.at[pl.ds(b * SCCH, SCCH)],
                sem_g).wait()
            pltpu.make_async_copy(
                kern_src(0), kv.at[pl.ds(b * SCCH, SCCH)], sem_g).wait()

        def start_idx(j, b):
            pltpu.async_copy(src_hbm.at[pl.ds(row_w + j, 1)],
                             siv.at[pl.ds(b, 1)], sem_i)
            pltpu.async_copy(dst_hbm.at[pl.ds(row_w + j, 1)],
                             div.at[pl.ds(b, 1)], sem_i)

        def wait_idx(b):
            pltpu.make_async_copy(src_hbm.at[pl.ds(row_w, 1)],
                                  siv.at[pl.ds(b, 1)], sem_i).wait()
            pltpu.make_async_copy(dst_hbm.at[pl.ds(row_w, 1)],
                                  div.at[pl.ds(b, 1)], sem_i).wait()

        def compute_scatter(b):
            def mulrow(r, carry):
                rr = b * SCCH + r
                for t in range(d // 16):
                    sl = pl.ds(t * 16, 16)
                    kv[rr, sl] = kv[rr, sl] * hv[rr, sl]
                return carry

            lax.fori_loop(0, SCCH, mulrow, 0, unroll=4)
            pltpu.sync_copy(kv.at[pl.ds(b * SCCH, SCCH)],
                            acc.at[div.at[b]], add=True)

        pltpu.sync_copy(src_hbm.at[pl.ds(row_w, 1)], siv.at[pl.ds(0, 1)])
        pltpu.sync_copy(dst_hbm.at[pl.ds(row_w, 1)], div.at[pl.ds(0, 1)])
        start_group(0, 0)

        npair = nch // 2

        def body(j2, carry):
            j = 2 * j2
            start_idx(j + 1, 1)
            wait_group(0)
            wait_idx(1)
            start_group(j + 1, 1)
            compute_scatter(0)
            @pl.when(j2 + 1 < npair)
            def _():
                start_idx(j + 2, 0)
                wait_group(1)
                wait_idx(0)
                start_group(j + 2, 0)
                compute_scatter(1)

            @pl.when(j2 + 1 >= npair)
            def _():
                wait_group(1)
                compute_scatter(1)

            return carry

        lax.fori_loop(0, npair, body, 0)
        plsc.subcore_barrier()

        def rb(j, carry):
            r = r0 + j * rbr
            pltpu.sync_copy(acc.at[pl.ds(r, rbr)], rbuf)
            pltpu.sync_copy(rbuf, out_hbm.at[cid].at[pl.ds(r, rbr)])
            return carry

        lax.fori_loop(0, rows_pt // rbr, rb, 0)

    return k(h_tab, kern, src2, dst2)


def _gms_dispatch(h, kern, src2, dst2, lvl, np_l):
    d = kern.shape[1]
    if np_l * d * 4 > 5_000_000:
        dh = d // 2
        parts = [_sc_gms(h[:, c:c + dh], kern, src2, dst2, lvl, np_l,
                         col0=c) for c in range(0, d, dh)]
        return jnp.concatenate(parts, axis=2)
    return _sc_gms(h, kern, src2, dst2, lvl, np_l)


def _sc_scatter_add(rows, idx, lvl, np_l):
    d = rows.shape[1]
    if np_l * d * 4 > 5_000_000 and d > 128:
        parts = [_sc_scatter(rows, idx, lvl, np_l, 128, col0=c, dfull=d)
                 for c in range(0, d, 128)]
        return jnp.concatenate(parts, axis=2)
    return _sc_scatter(rows, idx, lvl, np_l, d)


def _sc_scatter_ones(idx, lvl, np_l):
    return _sc_scatter(None, idx, lvl, np_l, 16, ones_mode=True)


def kernel(pos, seq, ori, domain, seq_emb, params, x, edge_index, batch):
    np0 = LVL_NP[0]
    src = edge_index[0].astype(I32)
    dst = edge_index[1].astype(I32)
    src_p = jnp.concatenate([src, jnp.zeros((EP - E,), I32)])
    dst_p = jnp.concatenate([dst, jnp.full((EP - E,), N0, I32)])
    s2 = src_p.reshape(EP, 1)
    d2 = dst_p.reshape(EP, 1)
    src_p = src_p.reshape(EP // 128, 128)
    dst_p = dst_p.reshape(EP // 128, 128)

    g_tab = jnp.concatenate(
        [pos, jnp.zeros((N0, 1), F32), ori.reshape(N0, 9),
         jnp.zeros((N0, 3), F32)], axis=1)
    g_tab = jnp.pad(g_tab, ((0, np0 - N0), (0, 0)))

    x2 = jnp.pad(x.astype(I32), (0, np0 - N0),
                 constant_values=31).reshape(np0, 1)
    emb32 = jnp.pad(params["emb"], ((0, 11), (0, 0)))
    h = _embed_call(x2, emb32, np0)

    g0 = g1 = c0 = c1 = None
    for lvl in range(4):
        n_l = LVL_N[lvl]
        np_l = LVL_NP[lvl]
        iA, iB = 2 * lvl, 2 * lvl + 1
        k1a = jnp.concatenate([params[f"b{iA}_k1"], jnp.zeros((3, 24), F32)])
        k1b = jnp.concatenate([params[f"b{iB}_k1"], jnp.zeros((3, 24), F32)])
        k2a = params[f"b{iA}_k2"]
        k2b = params[f"b{iB}_k2"]

        gs = _sc_gather(g_tab, src_p, lvl)
        gd = _sc_gather(g_tab, dst_p, lvl)
        degp = _sc_scatter_ones(dst_p, lvl, np_l)

        kern_a, kern_b = _kern_call(gs, gd, s2, d2,
                                    k1a, k2a, k1b, k2b, lvl)
        agg_a = _gms_dispatch(h, kern_a, src_p, dst_p, lvl, np_l)
        h = _node_update(agg_a, degp, h, params[f"b{iA}_lin"],
                         params[f"b{iA}_res"], n_l, np_l)

        agg_b = _gms_dispatch(h, kern_b, src_p, dst_p, lvl, np_l)
        h = _node_update(agg_b, degp, h, params[f"b{iB}_lin"],
                         params[f"b{iB}_res"], n_l, np_l)

        if lvl < 3:
            m = np_l // 2
            g_tab = _pool_call(g_tab.reshape(m, 2, 16))
            h = _pool_call(h.reshape(m, 2, h.shape[1]))

    batch_l = batch[::8].astype(I32)
    epg = 8192
    batch_p = jnp.pad(batch_l, (0, epg - LVL_N[3]),
                      constant_values=B).reshape(epg // 128, 128)
    h_p = jnp.pad(h, ((0, epg - LVL_NP[3]), (0, 0)))
    gpart = _sc_scatter_add(h_p, batch_p, 0, 128)
    cpart = _sc_scatter_ones(batch_p, 0, 128)

    wc1 = params["Wc1"]
    out = _classifier_call(
        gpart[0, :B], gpart[1, :B], cpart[0, :B], cpart[1, :B],
        seq_emb, domain, params["Ws"], params["Wq"], params["Wd"],
        wc1[0:256], wc1[256:512], wc1[512:768], params["Wc2"])
    return out


# speedup vs baseline: 7.0878x; 1.0014x over previous
import functools

import jax
import jax.numpy as jnp
from jax import lax
from jax.experimental import pallas as pl
from jax.experimental.pallas import tpu as pltpu
from jax.experimental.pallas import tpu_sc as plsc

F32 = jnp.float32
I32 = jnp.int32

N0 = 50000
E = 800000
EP = 819200
EC = 3200
NEB = EP // EC
NCH = 1600
LVL_N = [50000, 25000, 12500, 6250]
LVL_NP = [51200, 25600, 12800, 6400]
B = 64
SEQ_L = 5.0
IO_CH = [(16, 32), (32, 32), (32, 64), (64, 64),
         (64, 128), (128, 128), (128, 256), (256, 256)]


def _embed_call(x2, emb32, np0):
    def body(x_ref, emb_ref, out_ref):
        lane = lax.broadcasted_iota(I32, (NCH, 32), 1)
        oh = (lane == x_ref[...]).astype(F32)
        out_ref[...] = jnp.dot(oh, emb_ref[...], preferred_element_type=F32)

    return pl.pallas_call(
        body,
        grid=(np0 // NCH,),
        in_specs=[
            pl.BlockSpec((NCH, 1), lambda i: (i, 0)),
            pl.BlockSpec((32, 16), lambda i: (0, 0)),
        ],
        out_specs=pl.BlockSpec((NCH, 16), lambda i: (i, 0)),
        out_shape=jax.ShapeDtypeStruct((np0, 16), F32),
    )(x2, emb32)


def _kern_call(gs, gd, s2, d2, k1a, k2a, k1b, k2b, lvl):
    ciA = k2a.shape[1]
    ciB = k2b.shape[1]
    scale = float(2 ** lvl) / SEQ_L

    def body(gs_ref, gd_ref, s_ref, d_ref, k1a_ref, k2a_ref,
             k1b_ref, k2b_ref, ka_ref, kb_ref):
        gsv = gs_ref[...]
        gdv = gd_ref[...]
        sl = jnp.right_shift(s_ref[...], lvl)
        dl = jnp.right_shift(d_ref[...], lvl)
        rel = (dl - sl).astype(F32) * scale
        lane = lax.broadcasted_iota(I32, (1, 16), 1)
        base = jnp.where(lane < 3, gdv - gsv, gsv * gdv)
        geo = base + rel * (lane == 3).astype(F32)
        ka = jnp.maximum(
            jnp.dot(geo, k1a_ref[...], preferred_element_type=F32), 0.0)
        ka_ref[...] = jnp.dot(ka, k2a_ref[...], preferred_element_type=F32)
        kb = jnp.maximum(
            jnp.dot(geo, k1b_ref[...], preferred_element_type=F32), 0.0)
        kb_ref[...] = jnp.dot(kb, k2b_ref[...], preferred_element_type=F32)

    return pl.pallas_call(
        body,
        grid=(NEB,),
        in_specs=[
            pl.BlockSpec((EC, 16), lambda i: (i, 0)),
            pl.BlockSpec((EC, 16), lambda i: (i, 0)),
            pl.BlockSpec((EC, 1), lambda i: (i, 0)),
            pl.BlockSpec((EC, 1), lambda i: (i, 0)),
            pl.BlockSpec((16, 24), lambda i: (0, 0)),
            pl.BlockSpec((24, ciA), lambda i: (0, 0)),
            pl.BlockSpec((16, 24), lambda i: (0, 0)),
            pl.BlockSpec((24, ciB), lambda i: (0, 0)),
        ],
        out_specs=[
            pl.BlockSpec((EC, ciA), lambda i: (i, 0)),
            pl.BlockSpec((EC, ciB), lambda i: (i, 0)),
        ],
        out_shape=[
            jax.ShapeDtypeStruct((EP, ciA), F32),
            jax.ShapeDtypeStruct((EP, ciB), F32),
        ],
    )(gs, gd, s2, d2, k1a, k2a, k1b, k2b)


def _mult_call(hs, kern):
    ci = hs.shape[1]

    def body(hs_ref, k_ref, out_ref):
        out_ref[...] = hs_ref[...] * k_ref[...]

    return pl.pallas_call(
        body,
        grid=(NEB,),
        in_specs=[
            pl.BlockSpec((EC, ci), lambda i: (i, 0)),
            pl.BlockSpec((EC, ci), lambda i: (i, 0)),
        ],
        out_specs=pl.BlockSpec((EC, ci), lambda i: (i, 0)),
        out_shape=jax.ShapeDtypeStruct((EP, ci), F32),
    )(hs, kern)


def _node_update(aggp, degp, h, lin, res, n_l, np_l):
    ci = lin.shape[0]
    co = lin.shape[1]
    nsteps = np_l // NCH
    inv_n = 1.0 / float(n_l)

    def body1(p0_ref, p1_ref, d0_ref, d1_ref, lin_ref, z_ref, st_ref):
        i = pl.program_id(0)
        deg = d0_ref[...][:, 0:1] + d1_ref[...][:, 0:1]
        agg = (p0_ref[...] + p1_ref[...]) / jnp.maximum(deg, 1.0)
        z = jnp.dot(agg, lin_ref[...], preferred_element_type=F32)
        z_ref[...] = z
        rid = i * NCH + lax.broadcasted_iota(I32, (NCH, 1), 0)
        m = (rid < n_l).astype(F32)
        zm = z * m
        s1 = jnp.sum(zm, axis=0, keepdims=True)
        s2 = jnp.sum(zm * z, axis=0, keepdims=True)

        @pl.when(i == 0)
        def _():
            st_ref[...] = jnp.zeros_like(st_ref)

        st_ref[0:1, :] += s1
        st_ref[1:2, :] += s2

    z, st = pl.pallas_call(
        body1,
        grid=(nsteps,),
        in_specs=[
            pl.BlockSpec((NCH, ci), lambda i: (i, 0)),
            pl.BlockSpec((NCH, ci), lambda i: (i, 0)),
            pl.BlockSpec((NCH, 16), lambda i: (i, 0)),
            pl.BlockSpec((NCH, 16), lambda i: (i, 0)),
            pl.BlockSpec((ci, co), lambda i: (0, 0)),
        ],
        out_specs=[
            pl.BlockSpec((NCH, co), lambda i: (i, 0)),
            pl.BlockSpec((8, co), lambda i: (0, 0)),
        ],
        out_shape=[
            jax.ShapeDtypeStruct((np_l, co), F32),
            jax.ShapeDtypeStruct((8, co), F32),
        ],
    )(aggp[0], aggp[1], degp[0], degp[1], lin)

    def body2(z_ref, st_ref, h_ref, res_ref, out_ref):
        mean = st_ref[0:1, :] * inv_n
        var = st_ref[1:2, :] * inv_n - mean * mean
        std = jnp.sqrt(jnp.maximum(var, 0.0))
        zn = (z_ref[...] - mean) / (std + 1e-5)
        out_ref[...] = jnp.maximum(zn, 0.0) + jnp.dot(
            h_ref[...], res_ref[...], preferred_element_type=F32)

    return pl.pallas_call(
        body2,
        grid=(nsteps,),
        in_specs=[
            pl.BlockSpec((NCH, co), lambda i: (i, 0)),
            pl.BlockSpec((8, co), lambda i: (0, 0)),
            pl.BlockSpec((NCH, ci), lambda i: (i, 0)),
            pl.BlockSpec((ci, co), lambda i: (0, 0)),
        ],
        out_specs=pl.BlockSpec((NCH, co), lambda i: (i, 0)),
        out_shape=jax.ShapeDtypeStruct((np_l, co), F32),
    )(z, st, h, res)


def _pool_call(a3):
    m = a3.shape[0]
    d = a3.shape[2]

    def body(a_ref, out_ref):
        out_ref[...] = (a_ref[:, 0, :] + a_ref[:, 1, :]) * 0.5

    return pl.pallas_call(
        body,
        grid=(m // NCH,),
        in_specs=[pl.BlockSpec((NCH, 2, d), lambda i: (i, 0, 0))],
        out_specs=pl.BlockSpec((NCH, d), lambda i: (i, 0)),
        out_shape=jax.ShapeDtypeStruct((m, d), F32),
    )(a3)


def _classifier_call(g0, g1, c0, c1, seq_emb, domain, ws, wq, wd,
                     wc1a, wc1b, wc1c, wc2):
    def body(g0_ref, g1_ref, c0_ref, c1_ref, se_ref, dom_ref, ws_ref,
             wq_ref, wd_ref, a_ref, b_ref, c_ref, w2_ref, out_ref):
        cnt = c0_ref[...][:, 0:1] + c1_ref[...][:, 0:1]
        g = (g0_ref[...] + g1_ref[...]) / jnp.maximum(cnt, 1.0)
        struct = jnp.dot(g, ws_ref[...], preferred_element_type=F32)
        seqf = jnp.dot(se_ref[...], wq_ref[...], preferred_element_type=F32)
        dom = dom_ref[...]
        mask = jnp.sum(dom, axis=1, keepdims=True) != 0.0
        domf = jnp.where(mask,
                         jnp.dot(dom, wd_ref[...], preferred_element_type=F32),
                         0.0)
        hid = (jnp.dot(struct, a_ref[...], preferred_element_type=F32)
               + jnp.dot(seqf, b_ref[...], preferred_element_type=F32)
               + jnp.dot(domf, c_ref[...], preferred_element_type=F32))
        mean = jnp.mean(hid, axis=0, keepdims=True)
        var = jnp.mean(hid * hid, axis=0, keepdims=True) - mean * mean
        std = jnp.sqrt(jnp.maximum(var, 0.0))
        hid = jnp.maximum((hid - mean) / (std + 1e-5), 0.0)
        out_ref[...] = jnp.dot(hid, w2_ref[...], preferred_element_type=F32)

    nc = wc2.shape[1]
    return pl.pallas_call(
        body,
        out_shape=jax.ShapeDtypeStruct((B, nc), F32),
    )(g0, g1, c0, c1, seq_emb, domain, ws, wq, wd, wc1a, wc1b, wc1c, wc2)


NW = 32
SCCH = 128


def _shift_idx2(idx_v, lvl, g):
    if lvl:
        for r in range(g):
            for t in range(SCCH // 16):
                sl = pl.ds(t * 16, 16)
                idx_v[r, sl] = jnp.right_shift(idx_v[r, sl], lvl)


def _grp(d):
    return max(1, min(8, 262144 // (SCCH * d * 4)))


def _sc_gather(table, idx2, lvl):
    d = table.shape[1]
    ep = idx2.shape[0] * SCCH
    per_w = ep // NW
    g = _grp(d)
    while g > 1 and (per_w // (SCCH * g)) % 2:
        g //= 2
    ngrp = per_w // (SCCH * g)
    assert ngrp % 2 == 0
    mesh = plsc.VectorSubcoreMesh(core_axis_name="c", subcore_axis_name="s")

    @functools.partial(
        pl.kernel,
        out_type=jax.ShapeDtypeStruct((ep, d), F32),
        mesh=mesh,
        compiler_params=pltpu.CompilerParams(use_tc_tiling_on_sc=False),
        scratch_types=[
            pltpu.VMEM((2 * g, SCCH), I32),
            pltpu.VMEM((2 * g * SCCH, d), F32),
            pltpu.SemaphoreType.DMA,
            pltpu.SemaphoreType.DMA,
        ],
    )
    def k(table_hbm, idx_hbm, out_hbm, idx_v, rows_v, sem_i, sem_g):
        wid = lax.axis_index("s") * 2 + lax.axis_index("c")
        base_w = wid * per_w
        row_w = base_w // SCCH

        def shift_rows(b):
            if lvl:
                for r in range(g):
                    for t in range(SCCH // 16):
                        sl = pl.ds(t * 16, 16)
                        idx_v[b * g + r, sl] = jnp.right_shift(
                            idx_v[b * g + r, sl], lvl)

        def start_idx(j, b):
            pltpu.async_copy(idx_hbm.at[pl.ds(row_w + j * g, g)],
                             idx_v.at[pl.ds(b * g, g)], sem_i)

        def wait_idx(b):
            pltpu.make_async_copy(idx_hbm.at[pl.ds(row_w, g)],
                                  idx_v.at[pl.ds(b * g, g)], sem_i).wait()

        def start_group(b):
            shift_rows(b)
            for r in range(g):
                pltpu.async_copy(
                    table_hbm.at[idx_v.at[b * g + r]],
                    rows_v.at[pl.ds((b * g + r) * SCCH, SCCH)], sem_g)

        def wait_group(b):
            for r in range(g):
                pltpu.make_async_copy(
                    table_hbm.at[idx_v.at[b * g + r]],
                    rows_v.at[pl.ds((b * g + r) * SCCH, SCCH)],
                    sem_g).wait()

        def write_out(j, b):
            pltpu.sync_copy(
                rows_v.at[pl.ds(b * g * SCCH, g * SCCH)],
                out_hbm.at[pl.ds(base_w + j * g * SCCH, g * SCCH)])

        pltpu.sync_copy(idx_hbm.at[pl.ds(row_w, g)],
                        idx_v.at[pl.ds(0, g)])
        start_group(0)
        npair = ngrp // 2

        def body(j2, carry):
            j = 2 * j2
            start_idx(j + 1, 1)
            wait_group(0)
            wait_idx(1)
            start_group(1)
            write_out(j, 0)

            @pl.when(j2 + 1 < npair)
            def _():
                start_idx(j + 2, 0)
                wait_group(1)
                wait_idx(0)
                start_group(0)
                write_out(j + 1, 1)

            @pl.when(j2 + 1 >= npair)
            def _():
                wait_group(1)
                write_out(j + 1, 1)

            return carry

        lax.fori_loop(0, npair, body, 0)

    return k(table, idx2)


def _sc_scatter(rows, idx2, lvl, np_l, d, ones_mode=False, col0=0,
                dfull=None):
    if dfull is None:
        dfull = d
    ep = idx2.shape[0] * SCCH
    per_w = ep // NW
    g = 2 if ones_mode else 1
    ngrp = per_w // (SCCH * g)
    rows_pt = np_l // 16
    zr = 8
    rbr = min(400, max(8, 4096 // d))
    while rows_pt % rbr:
        rbr //= 2
    mesh = plsc.VectorSubcoreMesh(core_axis_name="c", subcore_axis_name="s")

    scratch = [
        pltpu.VMEM((g, SCCH), I32),
        pltpu.VMEM((g * SCCH, d), F32),
        pltpu.VMEM((zr, d), F32),
        pltpu.VMEM((rbr, d), F32),
        pltpu.VMEM_SHARED((np_l, d), F32),
        pltpu.SemaphoreType.DMA,
    ]

    def body_common(rows_hbm, idx_hbm, out_hbm, idx_v, rows_v, zbuf, rbuf,
                    acc, sem):
        cid = lax.axis_index("c")
        sid = lax.axis_index("s")
        wid = sid * 2 + cid
        for r in range(zr):
            for t in range(d // 16):
                zbuf[r, pl.ds(t * 16, 16)] = jnp.zeros((16,), F32)
        r0 = sid * rows_pt

        def zb(j, carry):
            pltpu.sync_copy(zbuf, acc.at[pl.ds(r0 + j * zr, zr)])
            return carry

        lax.fori_loop(0, rows_pt // zr, zb, 0)

        if ones_mode:
            for r in range(g * SCCH):
                for t in range(d // 16):
                    rows_v[r, pl.ds(t * 16, 16)] = jnp.ones((16,), F32)

        plsc.subcore_barrier()
        base_w = wid * per_w

        def body(j, carry):
            base = base_w + j * (SCCH * g)
            pltpu.sync_copy(idx_hbm.at[pl.ds(base // SCCH, g)], idx_v)
            _shift_idx2(idx_v, lvl, g)
            if not ones_mode:
                if d == dfull:
                    pltpu.sync_copy(rows_hbm.at[pl.ds(base, SCCH * g)],
                                    rows_v)
                else:
                    pltpu.sync_copy(
                        rows_hbm.at[pl.ds(base, SCCH * g), pl.ds(col0, d)],
                        rows_v)
            for r in range(g):
                pltpu.sync_copy(rows_v.at[pl.ds(r * SCCH, SCCH)],
                                acc.at[idx_v.at[r]], add=True)
            return carry

        lax.fori_loop(0, ngrp, body, 0)
        plsc.subcore_barrier()

        def rb(j, carry):
            r = r0 + j * rbr
            pltpu.sync_copy(acc.at[pl.ds(r, rbr)], rbuf)
            pltpu.sync_copy(rbuf, out_hbm.at[cid].at[pl.ds(r, rbr)])
            return carry

        lax.fori_loop(0, rows_pt // rbr, rb, 0)

    if ones_mode:
        @functools.partial(
            pl.kernel,
            out_type=jax.ShapeDtypeStruct((2, np_l, d), F32),
            mesh=mesh, scratch_types=scratch,
            compiler_params=pltpu.CompilerParams(use_tc_tiling_on_sc=False),
        )
        def k1(idx_hbm, out_hbm, idx_v, rows_v, zbuf, rbuf, acc, sem):
            body_common(None, idx_hbm, out_hbm, idx_v, rows_v, zbuf, rbuf,
                        acc, sem)

        return k1(idx2)

    @functools.partial(
        pl.kernel,
        out_type=jax.ShapeDtypeStruct((2, np_l, d), F32),
        mesh=mesh, scratch_types=scratch,
        compiler_params=pltpu.CompilerParams(use_tc_tiling_on_sc=False),
    )
    def k2(rows_hbm, idx_hbm, out_hbm, idx_v, rows_v, zbuf, rbuf, acc, sem):
        body_common(rows_hbm, idx_hbm, out_hbm, idx_v, rows_v, zbuf, rbuf,
                    acc, sem)

    return k2(rows, idx2)


def _sc_gms(h_tab, kern, src2, dst2, lvl, np_l, col0=0):
    d = h_tab.shape[1]
    dk = kern.shape[1]
    ep = src2.shape[0] * SCCH
    per_w = ep // NW
    nch = per_w // SCCH
    rows_pt = np_l // 16
    zr = 8
    rbr = min(400, max(8, 4096 // d))
    while rows_pt % rbr:
        rbr //= 2
    mesh = plsc.VectorSubcoreMesh(core_axis_name="c", subcore_axis_name="s")

    @functools.partial(
        pl.kernel,
        out_type=jax.ShapeDtypeStruct((2, np_l, d), F32),
        mesh=mesh,
        compiler_params=pltpu.CompilerParams(use_tc_tiling_on_sc=False),
        scratch_types=[
            pltpu.VMEM((2, SCCH), I32),
            pltpu.VMEM((2, SCCH), I32),
            pltpu.VMEM((2 * SCCH, d), F32),
            pltpu.VMEM((2 * SCCH, d), F32),
            pltpu.VMEM((zr, d), F32),
            pltpu.VMEM((rbr, d), F32),
            pltpu.VMEM_SHARED((np_l, d), F32),
            pltpu.SemaphoreType.DMA,
            pltpu.SemaphoreType.DMA,
        ],
    )
    def k(h_hbm, kern_hbm, src_hbm, dst_hbm, out_hbm, siv, div, hv, kv,
          zbuf, rbuf, acc, sem_i, sem_g):
        cid = lax.axis_index("c")
        sid = lax.axis_index("s")
        wid = sid * 2 + cid
        for r in range(zr):
            for t in range(d // 16):
                zbuf[r, pl.ds(t * 16, 16)] = jnp.zeros((16,), F32)
        r0 = sid * rows_pt

        def zb(j, carry):
            pltpu.sync_copy(zbuf, acc.at[pl.ds(r0 + j * zr, zr)])
            return carry

        lax.fori_loop(0, rows_pt // zr, zb, 0)
        plsc.subcore_barrier()
        base_w = wid * per_w
        row_w = base_w // SCCH

        def shift_row(ref, b):
            if lvl:
                for t in range(SCCH // 16):
                    sl = pl.ds(t * 16, 16)
                    ref[b, sl] = jnp.right_shift(ref[b, sl], lvl)

        def kern_src(j):
            if col0 == 0 and d == dk:
                return kern_hbm.at[pl.ds(base_w + j * SCCH, SCCH)]
            return kern_hbm.at[pl.ds(base_w + j * SCCH, SCCH),
                               pl.ds(col0, d)]

        def start_group(j, b):
            shift_row(siv, b)
            shift_row(div, b)
            pltpu.async_copy(h_hbm.at[siv.at[b]],
                             hv.at[pl.ds(b * SCCH, SCCH)], sem_g)
            pltpu.async_copy(kern_src(j), kv.at[pl.ds(b * SCCH, SCCH)],
                             sem_g)

        def wait_group(b):
            pltpu.make_async_copy(
                h_hbm.at[siv.at[b]], hv.at[pl.ds(b * SCCH, SCCH)],
                sem_g).wait()
            pltpu.make_async_copy(
                kern_src(0), kv.at[pl.ds(b * SCCH, SCCH)], sem_g).wait()

        def start_idx(j, b):
            pltpu.async_copy(src_hbm.at[pl.ds(row_w + j, 1)],
                             siv.at[pl.ds(b, 1)], sem_i)
            pltpu.async_copy(dst_hbm.at[pl.ds(row_w + j, 1)],
                             div.at[pl.ds(b, 1)], sem_i)

        def wait_idx(b):
            pltpu.make_async_copy(src_hbm.at[pl.ds(row_w, 1)],
                                  siv.at[pl.ds(b, 1)], sem_i).wait()
            pltpu.make_async_copy(dst_hbm.at[pl.ds(row_w, 1)],
                                  div.at[pl.ds(b, 1)], sem_i).wait()

        def compute_scatter(b):
            def mulrow(r, carry):
                rr = b * SCCH + r
                for t in range(d // 16):
                    sl = pl.ds(t * 16, 16)
                    kv[rr, sl] = kv[rr, sl] * hv[rr, sl]
                return carry

            lax.fori_loop(0, SCCH, mulrow, 0, unroll=4)
            pltpu.sync_copy(kv.at[pl.ds(b * SCCH, SCCH)],
                            acc.at[div.at[b]], add=True)

        pltpu.sync_copy(src_hbm.at[pl.ds(row_w, 1)], siv.at[pl.ds(0, 1)])
        pltpu.sync_copy(dst_hbm.at[pl.ds(row_w, 1)], div.at[pl.ds(0, 1)])
        start_group(0, 0)

        npair = nch // 2

        def body(j2, carry):
            j = 2 * j2
            start_idx(j + 1, 1)
            wait_group(0)
            wait_idx(1)
            start_group(j + 1, 1)
            compute_scatter(0)
            @pl.when(j2 + 1 < npair)
            def _():
                start_idx(j + 2, 0)
                wait_group(1)
                wait_idx(0)
                start_group(j + 2, 0)
                compute_scatter(1)

            @pl.when(j2 + 1 >= npair)
            def _():
                wait_group(1)
                compute_scatter(1)

            return carry

        lax.fori_loop(0, npair, body, 0)
        plsc.subcore_barrier()

        def rb(j, carry):
            r = r0 + j * rbr
            pltpu.sync_copy(acc.at[pl.ds(r, rbr)], rbuf)
            pltpu.sync_copy(rbuf, out_hbm.at[cid].at[pl.ds(r, rbr)])
            return carry

        lax.fori_loop(0, rows_pt // rbr, rb, 0)

    return k(h_tab, kern, src2, dst2)


def _gms_dispatch(h, kern, src2, dst2, lvl, np_l):
    d = kern.shape[1]
    if np_l * d * 4 > 5_000_000:
        dh = d // 2
        parts = [_sc_gms(h[:, c:c + dh], kern, src2, dst2, lvl, np_l,
                         col0=c) for c in range(0, d, dh)]
        return jnp.concatenate(parts, axis=2)
    return _sc_gms(h, kern, src2, dst2, lvl, np_l)


def _sc_scatter_add(rows, idx, lvl, np_l):
    d = rows.shape[1]
    if np_l * d * 4 > 5_000_000 and d > 128:
        parts = [_sc_scatter(rows, idx, lvl, np_l, 128, col0=c, dfull=d)
                 for c in range(0, d, 128)]
        return jnp.concatenate(parts, axis=2)
    return _sc_scatter(rows, idx, lvl, np_l, d)


def _sc_scatter_ones(idx, lvl, np_l):
    return _sc_scatter(None, idx, lvl, np_l, 16, ones_mode=True)


def kernel(pos, seq, ori, domain, seq_emb, params, x, edge_index, batch):
    np0 = LVL_NP[0]
    src = edge_index[0].astype(I32)
    dst = edge_index[1].astype(I32)
    src_p = jnp.concatenate([src, jnp.zeros((EP - E,), I32)])
    dst_p = jnp.concatenate([dst, jnp.full((EP - E,), N0, I32)])
    s2 = src_p.reshape(EP, 1)
    d2 = dst_p.reshape(EP, 1)
    src_p = src_p.reshape(EP // 128, 128)
    dst_p = dst_p.reshape(EP // 128, 128)

    g_tab = jnp.concatenate(
        [pos, jnp.zeros((N0, 1), F32), ori.reshape(N0, 9),
         jnp.zeros((N0, 3), F32)], axis=1)
    g_tab = jnp.pad(g_tab, ((0, np0 - N0), (0, 0)))

    x2 = jnp.pad(x.astype(I32), (0, np0 - N0),
                 constant_values=31).reshape(np0, 1)
    emb32 = jnp.pad(params["emb"], ((0, 11), (0, 0)))
    h = _embed_call(x2, emb32, np0)

    g0 = g1 = c0 = c1 = None
    for lvl in range(4):
        n_l = LVL_N[lvl]
        np_l = LVL_NP[lvl]
        iA, iB = 2 * lvl, 2 * lvl + 1
        k1a = jnp.concatenate([params[f"b{iA}_k1"], jnp.zeros((3, 24), F32)])
        k1b = jnp.concatenate([params[f"b{iB}_k1"], jnp.zeros((3, 24), F32)])
        k2a = params[f"b{iA}_k2"]
        k2b = params[f"b{iB}_k2"]

        gs = _sc_gather(g_tab, src_p, lvl)
        gd = _sc_gather(g_tab, dst_p, lvl)
        degp = _sc_scatter_ones(dst_p, lvl, np_l)

        kern_a, kern_b = _kern_call(gs, gd, s2, d2,
                                    k1a, k2a, k1b, k2b, lvl)
        agg_a = _gms_dispatch(h, kern_a, src_p, dst_p, lvl, np_l)
        h = _node_update(agg_a, degp, h, params[f"b{iA}_lin"],
                         params[f"b{iA}_res"], n_l, np_l)

        agg_b = _gms_dispatch(h, kern_b, src_p, dst_p, lvl, np_l)
        h = _node_update(agg_b, degp, h, params[f"b{iB}_lin"],
                         params[f"b{iB}_res"], n_l, np_l)

        if lvl < 3:
            m = np_l // 2
            g_tab = _pool_call(g_tab.reshape(m, 2, 16))
            h = _pool_call(h.reshape(m, 2, h.shape[1]))

    batch_l = batch[::8].astype(I32)
    epg = 8192
    batch_p = jnp.pad(batch_l, (0, epg - LVL_N[3]),
                      constant_values=B).reshape(epg // 128, 128)
    h_p = jnp.pad(h, ((0, epg - LVL_NP[3]), (0, 0)))
    gpart = _sc_scatter_add(h_p, batch_p, 0, 128)
    cpart = _sc_scatter_ones(batch_p, 0, 128)

    wc1 = params["Wc1"]
    out = _classifier_call(
        gpart[0, :B], gpart[1, :B], cpart[0, :B], cpart[1, :B],
        seq_emb, domain, params["Ws"], params["Wq"], params["Wd"],
        wc1[0:256], wc1[256:512], wc1[512:768], params["Wc2"])
    return out


# speedup vs baseline: 7.1865x; 1.0139x over previous
import functools

import jax
import jax.numpy as jnp
from jax import lax
from jax.experimental import pallas as pl
from jax.experimental.pallas import tpu as pltpu
from jax.experimental.pallas import tpu_sc as plsc

F32 = jnp.float32
I32 = jnp.int32

N0 = 50000
E = 800000
EP = 819200
EC = 3200
NEB = EP // EC
NCH = 1600
LVL_N = [50000, 25000, 12500, 6250]
LVL_NP = [51200, 25600, 12800, 6400]
B = 64
SEQ_L = 5.0
IO_CH = [(16, 32), (32, 32), (32, 64), (64, 64),
         (64, 128), (128, 128), (128, 256), (256, 256)]


def _embed_call(x2, emb32, np0):
    def body(x_ref, emb_ref, out_ref):
        lane = lax.broadcasted_iota(I32, (NCH, 32), 1)
        oh = (lane == x_ref[...]).astype(F32)
        out_ref[...] = jnp.dot(oh, emb_ref[...], preferred_element_type=F32)

    return pl.pallas_call(
        body,
        grid=(np0 // NCH,),
        in_specs=[
            pl.BlockSpec((NCH, 1), lambda i: (i, 0)),
            pl.BlockSpec((32, 16), lambda i: (0, 0)),
        ],
        out_specs=pl.BlockSpec((NCH, 16), lambda i: (i, 0)),
        out_shape=jax.ShapeDtypeStruct((np0, 16), F32),
    )(x2, emb32)


def _kern_call(gs, gd, s2, d2, k1a, k2a, k1b, k2b, lvl):
    ciA = k2a.shape[1]
    ciB = k2b.shape[1]
    scale = float(2 ** lvl) / SEQ_L

    def body(gs_ref, gd_ref, s_ref, d_ref, k1a_ref, k2a_ref,
             k1b_ref, k2b_ref, ka_ref, kb_ref):
        gsv = gs_ref[...]
        gdv = gd_ref[...]
        sl = jnp.right_shift(s_ref[...], lvl)
        dl = jnp.right_shift(d_ref[...], lvl)
        rel = (dl - sl).astype(F32) * scale
        lane = lax.broadcasted_iota(I32, (1, 16), 1)
        base = jnp.where(lane < 3, gdv - gsv, gsv * gdv)
        geo = base + rel * (lane == 3).astype(F32)
        ka = jnp.maximum(
            jnp.dot(geo, k1a_ref[...], preferred_element_type=F32), 0.0)
        ka_ref[...] = jnp.dot(ka, k2a_ref[...], preferred_element_type=F32)
        kb = jnp.maximum(
            jnp.dot(geo, k1b_ref[...], preferred_element_type=F32), 0.0)
        kb_ref[...] = jnp.dot(kb, k2b_ref[...], preferred_element_type=F32)

    return pl.pallas_call(
        body,
        grid=(NEB,),
        in_specs=[
            pl.BlockSpec((EC, 16), lambda i: (i, 0)),
            pl.BlockSpec((EC, 16), lambda i: (i, 0)),
            pl.BlockSpec((EC, 1), lambda i: (i, 0)),
            pl.BlockSpec((EC, 1), lambda i: (i, 0)),
            pl.BlockSpec((16, 24), lambda i: (0, 0)),
            pl.BlockSpec((24, ciA), lambda i: (0, 0)),
            pl.BlockSpec((16, 24), lambda i: (0, 0)),
            pl.BlockSpec((24, ciB), lambda i: (0, 0)),
        ],
        out_specs=[
            pl.BlockSpec((EC, ciA), lambda i: (i, 0)),
            pl.BlockSpec((EC, ciB), lambda i: (i, 0)),
        ],
        out_shape=[
            jax.ShapeDtypeStruct((EP, ciA), F32),
            jax.ShapeDtypeStruct((EP, ciB), F32),
        ],
    )(gs, gd, s2, d2, k1a, k2a, k1b, k2b)


def _mult_call(hs, kern):
    ci = hs.shape[1]

    def body(hs_ref, k_ref, out_ref):
        out_ref[...] = hs_ref[...] * k_ref[...]

    return pl.pallas_call(
        body,
        grid=(NEB,),
        in_specs=[
            pl.BlockSpec((EC, ci), lambda i: (i, 0)),
            pl.BlockSpec((EC, ci), lambda i: (i, 0)),
        ],
        out_specs=pl.BlockSpec((EC, ci), lambda i: (i, 0)),
        out_shape=jax.ShapeDtypeStruct((EP, ci), F32),
    )(hs, kern)


def _node_update(aggp, degp, h, lin, res, n_l, np_l):
    ci = lin.shape[0]
    co = lin.shape[1]
    nsteps = np_l // NCH
    inv_n = 1.0 / float(n_l)

    def body1(p0_ref, p1_ref, d0_ref, d1_ref, lin_ref, z_ref, st_ref):
        i = pl.program_id(0)
        deg = d0_ref[...][:, 0:1] + d1_ref[...][:, 0:1]
        agg = (p0_ref[...] + p1_ref[...]) / jnp.maximum(deg, 1.0)
        z = jnp.dot(agg, lin_ref[...], preferred_element_type=F32)
        z_ref[...] = z
        rid = i * NCH + lax.broadcasted_iota(I32, (NCH, 1), 0)
        m = (rid < n_l).astype(F32)
        zm = z * m
        s1 = jnp.sum(zm, axis=0, keepdims=True)
        s2 = jnp.sum(zm * z, axis=0, keepdims=True)

        @pl.when(i == 0)
        def _():
            st_ref[...] = jnp.zeros_like(st_ref)

        st_ref[0:1, :] += s1
        st_ref[1:2, :] += s2

    z, st = pl.pallas_call(
        body1,
        grid=(nsteps,),
        in_specs=[
            pl.BlockSpec((NCH, ci), lambda i: (i, 0)),
            pl.BlockSpec((NCH, ci), lambda i: (i, 0)),
            pl.BlockSpec((NCH, 16), lambda i: (i, 0)),
            pl.BlockSpec((NCH, 16), lambda i: (i, 0)),
            pl.BlockSpec((ci, co), lambda i: (0, 0)),
        ],
        out_specs=[
            pl.BlockSpec((NCH, co), lambda i: (i, 0)),
            pl.BlockSpec((8, co), lambda i: (0, 0)),
        ],
        out_shape=[
            jax.ShapeDtypeStruct((np_l, co), F32),
            jax.ShapeDtypeStruct((8, co), F32),
        ],
    )(aggp[0], aggp[1], degp[0], degp[1], lin)

    def body2(z_ref, st_ref, h_ref, res_ref, out_ref):
        mean = st_ref[0:1, :] * inv_n
        var = st_ref[1:2, :] * inv_n - mean * mean
        std = jnp.sqrt(jnp.maximum(var, 0.0))
        zn = (z_ref[...] - mean) / (std + 1e-5)
        out_ref[...] = jnp.maximum(zn, 0.0) + jnp.dot(
            h_ref[...], res_ref[...], preferred_element_type=F32)

    return pl.pallas_call(
        body2,
        grid=(nsteps,),
        in_specs=[
            pl.BlockSpec((NCH, co), lambda i: (i, 0)),
            pl.BlockSpec((8, co), lambda i: (0, 0)),
            pl.BlockSpec((NCH, ci), lambda i: (i, 0)),
            pl.BlockSpec((ci, co), lambda i: (0, 0)),
        ],
        out_specs=pl.BlockSpec((NCH, co), lambda i: (i, 0)),
        out_shape=jax.ShapeDtypeStruct((np_l, co), F32),
    )(z, st, h, res)


def _pool_call(a3):
    m = a3.shape[0]
    d = a3.shape[2]

    def body(a_ref, out_ref):
        out_ref[...] = (a_ref[:, 0, :] + a_ref[:, 1, :]) * 0.5

    return pl.pallas_call(
        body,
        grid=(m // NCH,),
        in_specs=[pl.BlockSpec((NCH, 2, d), lambda i: (i, 0, 0))],
        out_specs=pl.BlockSpec((NCH, d), lambda i: (i, 0)),
        out_shape=jax.ShapeDtypeStruct((m, d), F32),
    )(a3)


def _classifier_call(g0, g1, c0, c1, seq_emb, domain, ws, wq, wd,
                     wc1a, wc1b, wc1c, wc2):
    def body(g0_ref, g1_ref, c0_ref, c1_ref, se_ref, dom_ref, ws_ref,
             wq_ref, wd_ref, a_ref, b_ref, c_ref, w2_ref, out_ref):
        cnt = c0_ref[...][:, 0:1] + c1_ref[...][:, 0:1]
        g = (g0_ref[...] + g1_ref[...]) / jnp.maximum(cnt, 1.0)
        struct = jnp.dot(g, ws_ref[...], preferred_element_type=F32)
        seqf = jnp.dot(se_ref[...], wq_ref[...], preferred_element_type=F32)
        dom = dom_ref[...]
        mask = jnp.sum(dom, axis=1, keepdims=True) != 0.0
        domf = jnp.where(mask,
                         jnp.dot(dom, wd_ref[...], preferred_element_type=F32),
                         0.0)
        hid = (jnp.dot(struct, a_ref[...], preferred_element_type=F32)
               + jnp.dot(seqf, b_ref[...], preferred_element_type=F32)
               + jnp.dot(domf, c_ref[...], preferred_element_type=F32))
        mean = jnp.mean(hid, axis=0, keepdims=True)
        var = jnp.mean(hid * hid, axis=0, keepdims=True) - mean * mean
        std = jnp.sqrt(jnp.maximum(var, 0.0))
        hid = jnp.maximum((hid - mean) / (std + 1e-5), 0.0)
        out_ref[...] = jnp.dot(hid, w2_ref[...], preferred_element_type=F32)

    nc = wc2.shape[1]
    return pl.pallas_call(
        body,
        out_shape=jax.ShapeDtypeStruct((B, nc), F32),
    )(g0, g1, c0, c1, seq_emb, domain, ws, wq, wd, wc1a, wc1b, wc1c, wc2)


NW = 32
SCCH = 128


def _shift_idx2(idx_v, lvl, g):
    if lvl:
        for r in range(g):
            for t in range(SCCH // 16):
                sl = pl.ds(t * 16, 16)
                idx_v[r, sl] = jnp.right_shift(idx_v[r, sl], lvl)


def _grp(d):
    return max(1, min(8, 262144 // (SCCH * d * 4)))


def _sc_gather(table, idx2, lvl):
    d = table.shape[1]
    ep = idx2.shape[0] * SCCH
    per_w = ep // NW
    g = _grp(d)
    while g > 1 and (per_w // (SCCH * g)) % 2:
        g //= 2
    ngrp = per_w // (SCCH * g)
    assert ngrp % 2 == 0
    mesh = plsc.VectorSubcoreMesh(core_axis_name="c", subcore_axis_name="s")

    @functools.partial(
        pl.kernel,
        out_type=jax.ShapeDtypeStruct((ep, d), F32),
        mesh=mesh,
        compiler_params=pltpu.CompilerParams(use_tc_tiling_on_sc=False),
        scratch_types=[
            pltpu.VMEM((2 * g, SCCH), I32),
            pltpu.VMEM((2 * g * SCCH, d), F32),
            pltpu.SemaphoreType.DMA,
            pltpu.SemaphoreType.DMA,
        ],
    )
    def k(table_hbm, idx_hbm, out_hbm, idx_v, rows_v, sem_i, sem_g):
        wid = lax.axis_index("s") * 2 + lax.axis_index("c")
        base_w = wid * per_w
        row_w = base_w // SCCH

        def shift_rows(b):
            if lvl:
                for r in range(g):
                    for t in range(SCCH // 16):
                        sl = pl.ds(t * 16, 16)
                        idx_v[b * g + r, sl] = jnp.right_shift(
                            idx_v[b * g + r, sl], lvl)

        def start_idx(j, b):
            pltpu.async_copy(idx_hbm.at[pl.ds(row_w + j * g, g)],
                             idx_v.at[pl.ds(b * g, g)], sem_i)

        def wait_idx(b):
            pltpu.make_async_copy(idx_hbm.at[pl.ds(row_w, g)],
                                  idx_v.at[pl.ds(b * g, g)], sem_i).wait()

        def start_group(b):
            shift_rows(b)
            for r in range(g):
                pltpu.async_copy(
                    table_hbm.at[idx_v.at[b * g + r]],
                    rows_v.at[pl.ds((b * g + r) * SCCH, SCCH)], sem_g)

        def wait_group(b):
            for r in range(g):
                pltpu.make_async_copy(
                    table_hbm.at[idx_v.at[b * g + r]],
                    rows_v.at[pl.ds((b * g + r) * SCCH, SCCH)],
                    sem_g).wait()

        def write_out(j, b):
            pltpu.sync_copy(
                rows_v.at[pl.ds(b * g * SCCH, g * SCCH)],
                out_hbm.at[pl.ds(base_w + j * g * SCCH, g * SCCH)])

        pltpu.sync_copy(idx_hbm.at[pl.ds(row_w, g)],
                        idx_v.at[pl.ds(0, g)])
        start_group(0)
        npair = ngrp // 2

        def body(j2, carry):
            j = 2 * j2
            start_idx(j + 1, 1)
            wait_group(0)
            wait_idx(1)
            start_group(1)
            write_out(j, 0)

            @pl.when(j2 + 1 < npair)
            def _():
                start_idx(j + 2, 0)
                wait_group(1)
                wait_idx(0)
                start_group(0)
                write_out(j + 1, 1)

            @pl.when(j2 + 1 >= npair)
            def _():
                wait_group(1)
                write_out(j + 1, 1)

            return carry

        lax.fori_loop(0, npair, body, 0)

    return k(table, idx2)


def _sc_scatter(rows, idx2, lvl, np_l, d, ones_mode=False, col0=0,
                dfull=None):
    if dfull is None:
        dfull = d
    ep = idx2.shape[0] * SCCH
    per_w = ep // NW
    g = 2 if ones_mode else 1
    ngrp = per_w // (SCCH * g)
    rows_pt = np_l // 16
    zr = 8
    rbr = min(400, max(8, 4096 // d))
    while rows_pt % rbr:
        rbr //= 2
    mesh = plsc.VectorSubcoreMesh(core_axis_name="c", subcore_axis_name="s")

    scratch = [
        pltpu.VMEM((g, SCCH), I32),
        pltpu.VMEM((g * SCCH, d), F32),
        pltpu.VMEM((zr, d), F32),
        pltpu.VMEM((rbr, d), F32),
        pltpu.VMEM_SHARED((np_l, d), F32),
        pltpu.SemaphoreType.DMA,
    ]

    def body_common(rows_hbm, idx_hbm, out_hbm, idx_v, rows_v, zbuf, rbuf,
                    acc, sem):
        cid = lax.axis_index("c")
        sid = lax.axis_index("s")
        wid = sid * 2 + cid
        for r in range(zr):
            for t in range(d // 16):
                zbuf[r, pl.ds(t * 16, 16)] = jnp.zeros((16,), F32)
        r0 = sid * rows_pt

        def zb(j, carry):
            pltpu.sync_copy(zbuf, acc.at[pl.ds(r0 + j * zr, zr)])
            return carry

        lax.fori_loop(0, rows_pt // zr, zb, 0)

        if ones_mode:
            for r in range(g * SCCH):
                for t in range(d // 16):
                    rows_v[r, pl.ds(t * 16, 16)] = jnp.ones((16,), F32)

        plsc.subcore_barrier()
        base_w = wid * per_w

        def body(j, carry):
            base = base_w + j * (SCCH * g)
            pltpu.sync_copy(idx_hbm.at[pl.ds(base // SCCH, g)], idx_v)
            _shift_idx2(idx_v, lvl, g)
            if not ones_mode:
                if d == dfull:
                    pltpu.sync_copy(rows_hbm.at[pl.ds(base, SCCH * g)],
                                    rows_v)
                else:
                    pltpu.sync_copy(
                        rows_hbm.at[pl.ds(base, SCCH * g), pl.ds(col0, d)],
                        rows_v)
            for r in range(g):
                pltpu.sync_copy(rows_v.at[pl.ds(r * SCCH, SCCH)],
                                acc.at[idx_v.at[r]], add=True)
            return carry

        lax.fori_loop(0, ngrp, body, 0)
        plsc.subcore_barrier()

        def rb(j, carry):
            r = r0 + j * rbr
            pltpu.sync_copy(acc.at[pl.ds(r, rbr)], rbuf)
            pltpu.sync_copy(rbuf, out_hbm.at[cid].at[pl.ds(r, rbr)])
            return carry

        lax.fori_loop(0, rows_pt // rbr, rb, 0)

    if ones_mode:
        @functools.partial(
            pl.kernel,
            out_type=jax.ShapeDtypeStruct((2, np_l, d), F32),
            mesh=mesh, scratch_types=scratch,
            compiler_params=pltpu.CompilerParams(use_tc_tiling_on_sc=False),
        )
        def k1(idx_hbm, out_hbm, idx_v, rows_v, zbuf, rbuf, acc, sem):
            body_common(None, idx_hbm, out_hbm, idx_v, rows_v, zbuf, rbuf,
                        acc, sem)

        return k1(idx2)

    @functools.partial(
        pl.kernel,
        out_type=jax.ShapeDtypeStruct((2, np_l, d), F32),
        mesh=mesh, scratch_types=scratch,
        compiler_params=pltpu.CompilerParams(use_tc_tiling_on_sc=False),
    )
    def k2(rows_hbm, idx_hbm, out_hbm, idx_v, rows_v, zbuf, rbuf, acc, sem):
        body_common(rows_hbm, idx_hbm, out_hbm, idx_v, rows_v, zbuf, rbuf,
                    acc, sem)

    return k2(rows, idx2)


def _sc_gms(h_tab, kern, src2, dst2, lvl, np_l, col0=0):
    d = h_tab.shape[1]
    dk = kern.shape[1]
    ep = src2.shape[0] * SCCH
    per_w = ep // NW
    nch = per_w // SCCH
    gg = 2 if d <= 64 else 1
    nst = nch // gg
    rows_pt = np_l // 16
    zr = 8
    rbr = min(400, max(8, 4096 // d))
    while rows_pt % rbr:
        rbr //= 2
    mesh = plsc.VectorSubcoreMesh(core_axis_name="c", subcore_axis_name="s")

    @functools.partial(
        pl.kernel,
        out_type=jax.ShapeDtypeStruct((2, np_l, d), F32),
        mesh=mesh,
        compiler_params=pltpu.CompilerParams(use_tc_tiling_on_sc=False),
        scratch_types=[
            pltpu.VMEM((2 * gg, SCCH), I32),
            pltpu.VMEM((2 * gg, SCCH), I32),
            pltpu.VMEM((2 * gg * SCCH, d), F32),
            pltpu.VMEM((2 * gg * SCCH, d), F32),
            pltpu.VMEM((zr, d), F32),
            pltpu.VMEM((rbr, d), F32),
            pltpu.VMEM_SHARED((np_l, d), F32),
            pltpu.SemaphoreType.DMA,
            pltpu.SemaphoreType.DMA,
        ],
    )
    def k(h_hbm, kern_hbm, src_hbm, dst_hbm, out_hbm, siv, div, hv, kv,
          zbuf, rbuf, acc, sem_i, sem_g):
        cid = lax.axis_index("c")
        sid = lax.axis_index("s")
        wid = sid * 2 + cid
        for r in range(zr):
            for t in range(d // 16):
                zbuf[r, pl.ds(t * 16, 16)] = jnp.zeros((16,), F32)
        r0 = sid * rows_pt

        def zb(j, carry):
            pltpu.sync_copy(zbuf, acc.at[pl.ds(r0 + j * zr, zr)])
            return carry

        lax.fori_loop(0, rows_pt // zr, zb, 0)
        plsc.subcore_barrier()
        base_w = wid * per_w
        row_w = base_w // SCCH

        def shift_rows(b):
            if lvl:
                for r in range(gg):
                    for t in range(SCCH // 16):
                        sl = pl.ds(t * 16, 16)
                        rr = b * gg + r
                        siv[rr, sl] = jnp.right_shift(siv[rr, sl], lvl)
                        div[rr, sl] = jnp.right_shift(div[rr, sl], lvl)

        def kern_src(st, r):
            base = base_w + (st * gg + r) * SCCH
            if col0 == 0 and d == dk:
                return kern_hbm.at[pl.ds(base, SCCH)]
            return kern_hbm.at[pl.ds(base, SCCH), pl.ds(col0, d)]

        def start_group(st, b):
            shift_rows(b)
            for r in range(gg):
                rr = b * gg + r
                pltpu.async_copy(h_hbm.at[siv.at[rr]],
                                 hv.at[pl.ds(rr * SCCH, SCCH)], sem_g)
                pltpu.async_copy(kern_src(st, r),
                                 kv.at[pl.ds(rr * SCCH, SCCH)], sem_g)

        def wait_group(b):
            for r in range(gg):
                rr = b * gg + r
                pltpu.make_async_copy(
                    h_hbm.at[siv.at[rr]], hv.at[pl.ds(rr * SCCH, SCCH)],
                    sem_g).wait()
                pltpu.make_async_copy(
                    kern_src(0, r), kv.at[pl.ds(rr * SCCH, SCCH)],
                    sem_g).wait()

        def start_idx(st, b):
            pltpu.async_copy(src_hbm.at[pl.ds(row_w + st * gg, gg)],
                             siv.at[pl.ds(b * gg, gg)], sem_i)
            pltpu.async_copy(dst_hbm.at[pl.ds(row_w + st * gg, gg)],
                             div.at[pl.ds(b * gg, gg)], sem_i)

        def wait_idx(b):
            pltpu.make_async_copy(src_hbm.at[pl.ds(row_w, gg)],
                                  siv.at[pl.ds(b * gg, gg)], sem_i).wait()
            pltpu.make_async_copy(dst_hbm.at[pl.ds(row_w, gg)],
                                  div.at[pl.ds(b * gg, gg)], sem_i).wait()

        def compute_scatter(b):
            for r in range(gg):
                rr = b * gg + r

                def mulrow(q, carry):
                    row = rr * SCCH + q
                    for t in range(d // 16):
                        sl = pl.ds(t * 16, 16)
                        kv[row, sl] = kv[row, sl] * hv[row, sl]
                    return carry

                lax.fori_loop(0, SCCH, mulrow, 0, unroll=4)
                pltpu.sync_copy(kv.at[pl.ds(rr * SCCH, SCCH)],
                                acc.at[div.at[rr]], add=True)

        pltpu.sync_copy(src_hbm.at[pl.ds(row_w, gg)],
                        siv.at[pl.ds(0, gg)])
        pltpu.sync_copy(dst_hbm.at[pl.ds(row_w, gg)],
                        div.at[pl.ds(0, gg)])
        start_group(0, 0)
        npair = nst // 2

        def body(j2, carry):
            st = 2 * j2
            start_idx(st + 1, 1)
            wait_group(0)
            wait_idx(1)
            start_group(st + 1, 1)
            compute_scatter(0)

            @pl.when(j2 + 1 < npair)
            def _():
                start_idx(st + 2, 0)
                wait_group(1)
                wait_idx(0)
                start_group(st + 2, 0)
                compute_scatter(1)

            @pl.when(j2 + 1 >= npair)
            def _():
                wait_group(1)
                compute_scatter(1)

            return carry

        lax.fori_loop(0, npair, body, 0)
        plsc.subcore_barrier()

        def rb(j, carry):
            r = r0 + j * rbr
            pltpu.sync_copy(acc.at[pl.ds(r, rbr)], rbuf)
            pltpu.sync_copy(rbuf, out_hbm.at[cid].at[pl.ds(r, rbr)])
            return carry

        lax.fori_loop(0, rows_pt // rbr, rb, 0)

    return k(h_tab, kern, src2, dst2)


def _gms_dispatch(h, kern, src2, dst2, lvl, np_l):
    d = kern.shape[1]
    if np_l * d * 4 > 5_000_000:
        dh = d // 2
        parts = [_sc_gms(h[:, c:c + dh], kern, src2, dst2, lvl, np_l,
                         col0=c) for c in range(0, d, dh)]
        return jnp.concatenate(parts, axis=2)
    return _sc_gms(h, kern, src2, dst2, lvl, np_l)


def _sc_scatter_add(rows, idx, lvl, np_l):
    d = rows.shape[1]
    if np_l * d * 4 > 5_000_000 and d > 128:
        parts = [_sc_scatter(rows, idx, lvl, np_l, 128, col0=c, dfull=d)
                 for c in range(0, d, 128)]
        return jnp.concatenate(parts, axis=2)
    return _sc_scatter(rows, idx, lvl, np_l, d)


def _sc_scatter_ones(idx, lvl, np_l):
    return _sc_scatter(None, idx, lvl, np_l, 16, ones_mode=True)


def kernel(pos, seq, ori, domain, seq_emb, params, x, edge_index, batch):
    np0 = LVL_NP[0]
    src = edge_index[0].astype(I32)
    dst = edge_index[1].astype(I32)
    src_p = jnp.concatenate([src, jnp.zeros((EP - E,), I32)])
    dst_p = jnp.concatenate([dst, jnp.full((EP - E,), N0, I32)])
    s2 = src_p.reshape(EP, 1)
    d2 = dst_p.reshape(EP, 1)
    src_p = src_p.reshape(EP // 128, 128)
    dst_p = dst_p.reshape(EP // 128, 128)

    g_tab = jnp.concatenate(
        [pos, jnp.zeros((N0, 1), F32), ori.reshape(N0, 9),
         jnp.zeros((N0, 3), F32)], axis=1)
    g_tab = jnp.pad(g_tab, ((0, np0 - N0), (0, 0)))

    x2 = jnp.pad(x.astype(I32), (0, np0 - N0),
                 constant_values=31).reshape(np0, 1)
    emb32 = jnp.pad(params["emb"], ((0, 11), (0, 0)))
    h = _embed_call(x2, emb32, np0)

    g0 = g1 = c0 = c1 = None
    for lvl in range(4):
        n_l = LVL_N[lvl]
        np_l = LVL_NP[lvl]
        iA, iB = 2 * lvl, 2 * lvl + 1
        k1a = jnp.concatenate([params[f"b{iA}_k1"], jnp.zeros((3, 24), F32)])
        k1b = jnp.concatenate([params[f"b{iB}_k1"], jnp.zeros((3, 24), F32)])
        k2a = params[f"b{iA}_k2"]
        k2b = params[f"b{iB}_k2"]

        gs = _sc_gather(g_tab, src_p, lvl)
        gd = _sc_gather(g_tab, dst_p, lvl)
        degp = _sc_scatter_ones(dst_p, lvl, np_l)

        kern_a, kern_b = _kern_call(gs, gd, s2, d2,
                                    k1a, k2a, k1b, k2b, lvl)
        agg_a = _gms_dispatch(h, kern_a, src_p, dst_p, lvl, np_l)
        h = _node_update(agg_a, degp, h, params[f"b{iA}_lin"],
                         params[f"b{iA}_res"], n_l, np_l)

        agg_b = _gms_dispatch(h, kern_b, src_p, dst_p, lvl, np_l)
        h = _node_update(agg_b, degp, h, params[f"b{iB}_lin"],
                         params[f"b{iB}_res"], n_l, np_l)

        if lvl < 3:
            m = np_l // 2
            g_tab = _pool_call(g_tab.reshape(m, 2, 16))
            h = _pool_call(h.reshape(m, 2, h.shape[1]))

    batch_l = batch[::8].astype(I32)
    epg = 8192
    batch_p = jnp.pad(batch_l, (0, epg - LVL_N[3]),
                      constant_values=B).reshape(epg // 128, 128)
    h_p = jnp.pad(h, ((0, epg - LVL_NP[3]), (0, 0)))
    gpart = _sc_scatter_add(h_p, batch_p, 0, 128)
    cpart = _sc_scatter_ones(batch_p, 0, 128)

    wc1 = params["Wc1"]
    out = _classifier_call(
        gpart[0, :B], gpart[1, :B], cpart[0, :B], cpart[1, :B],
        seq_emb, domain, params["Ws"], params["Wq"], params["Wd"],
        wc1[0:256], wc1[256:512], wc1[512:768], params["Wc2"])
    return out


# speedup vs baseline: 7.1962x; 1.0014x over previous
import functools

import jax
import jax.numpy as jnp
from jax import lax
from jax.experimental import pallas as pl
from jax.experimental.pallas import tpu as pltpu
from jax.experimental.pallas import tpu_sc as plsc

F32 = jnp.float32
I32 = jnp.int32

N0 = 50000
E = 800000
EP = 819200
EC = 3200
NEB = EP // EC
NCH = 1600
LVL_N = [50000, 25000, 12500, 6250]
LVL_NP = [51200, 25600, 12800, 6400]
B = 64
SEQ_L = 5.0
IO_CH = [(16, 32), (32, 32), (32, 64), (64, 64),
         (64, 128), (128, 128), (128, 256), (256, 256)]


def _embed_call(x2, emb32, np0):
    def body(x_ref, emb_ref, out_ref):
        lane = lax.broadcasted_iota(I32, (NCH, 32), 1)
        oh = (lane == x_ref[...]).astype(F32)
        out_ref[...] = jnp.dot(oh, emb_ref[...], preferred_element_type=F32)

    return pl.pallas_call(
        body,
        grid=(np0 // NCH,),
        in_specs=[
            pl.BlockSpec((NCH, 1), lambda i: (i, 0)),
            pl.BlockSpec((32, 16), lambda i: (0, 0)),
        ],
        out_specs=pl.BlockSpec((NCH, 16), lambda i: (i, 0)),
        out_shape=jax.ShapeDtypeStruct((np0, 16), F32),
    )(x2, emb32)


def _kern_call(gs, gd, s2, d2, k1a, k2a, k1b, k2b, lvl):
    ciA = k2a.shape[1]
    ciB = k2b.shape[1]
    scale = float(2 ** lvl) / SEQ_L

    def body(gs_ref, gd_ref, s_ref, d_ref, k1a_ref, k2a_ref,
             k1b_ref, k2b_ref, ka_ref, kb_ref):
        gsv = gs_ref[...]
        gdv = gd_ref[...]
        sl = jnp.right_shift(s_ref[...], lvl)
        dl = jnp.right_shift(d_ref[...], lvl)
        rel = (dl - sl).astype(F32) * scale
        lane = lax.broadcasted_iota(I32, (1, 16), 1)
        base = jnp.where(lane < 3, gdv - gsv, gsv * gdv)
        geo = base + rel * (lane == 3).astype(F32)
        ka = jnp.maximum(
            jnp.dot(geo, k1a_ref[...], preferred_element_type=F32), 0.0)
        ka_ref[...] = jnp.dot(ka, k2a_ref[...], preferred_element_type=F32)
        kb = jnp.maximum(
            jnp.dot(geo, k1b_ref[...], preferred_element_type=F32), 0.0)
        kb_ref[...] = jnp.dot(kb, k2b_ref[...], preferred_element_type=F32)

    return pl.pallas_call(
        body,
        grid=(NEB,),
        in_specs=[
            pl.BlockSpec((EC, 16), lambda i: (i, 0)),
            pl.BlockSpec((EC, 16), lambda i: (i, 0)),
            pl.BlockSpec((EC, 1), lambda i: (i, 0)),
            pl.BlockSpec((EC, 1), lambda i: (i, 0)),
            pl.BlockSpec((16, 24), lambda i: (0, 0)),
            pl.BlockSpec((24, ciA), lambda i: (0, 0)),
            pl.BlockSpec((16, 24), lambda i: (0, 0)),
            pl.BlockSpec((24, ciB), lambda i: (0, 0)),
        ],
        out_specs=[
            pl.BlockSpec((EC, ciA), lambda i: (i, 0)),
            pl.BlockSpec((EC, ciB), lambda i: (i, 0)),
        ],
        out_shape=[
            jax.ShapeDtypeStruct((EP, ciA), F32),
            jax.ShapeDtypeStruct((EP, ciB), F32),
        ],
    )(gs, gd, s2, d2, k1a, k2a, k1b, k2b)


def _mult_call(hs, kern):
    ci = hs.shape[1]

    def body(hs_ref, k_ref, out_ref):
        out_ref[...] = hs_ref[...] * k_ref[...]

    return pl.pallas_call(
        body,
        grid=(NEB,),
        in_specs=[
            pl.BlockSpec((EC, ci), lambda i: (i, 0)),
            pl.BlockSpec((EC, ci), lambda i: (i, 0)),
        ],
        out_specs=pl.BlockSpec((EC, ci), lambda i: (i, 0)),
        out_shape=jax.ShapeDtypeStruct((EP, ci), F32),
    )(hs, kern)


def _node_update(aggp, degp, h, lin, res, n_l, np_l):
    ci = lin.shape[0]
    co = lin.shape[1]
    nsteps = np_l // NCH
    inv_n = 1.0 / float(n_l)

    def body1(p0_ref, p1_ref, d0_ref, d1_ref, lin_ref, z_ref, st_ref):
        i = pl.program_id(0)
        deg = d0_ref[...][:, 0:1] + d1_ref[...][:, 0:1]
        agg = (p0_ref[...] + p1_ref[...]) / jnp.maximum(deg, 1.0)
        z = jnp.dot(agg, lin_ref[...], preferred_element_type=F32)
        z_ref[...] = z
        rid = i * NCH + lax.broadcasted_iota(I32, (NCH, 1), 0)
        m = (rid < n_l).astype(F32)
        zm = z * m
        s1 = jnp.sum(zm, axis=0, keepdims=True)
        s2 = jnp.sum(zm * z, axis=0, keepdims=True)

        @pl.when(i == 0)
        def _():
            st_ref[...] = jnp.zeros_like(st_ref)

        st_ref[0:1, :] += s1
        st_ref[1:2, :] += s2

    z, st = pl.pallas_call(
        body1,
        grid=(nsteps,),
        in_specs=[
            pl.BlockSpec((NCH, ci), lambda i: (i, 0)),
            pl.BlockSpec((NCH, ci), lambda i: (i, 0)),
            pl.BlockSpec((NCH, 16), lambda i: (i, 0)),
            pl.BlockSpec((NCH, 16), lambda i: (i, 0)),
            pl.BlockSpec((ci, co), lambda i: (0, 0)),
        ],
        out_specs=[
            pl.BlockSpec((NCH, co), lambda i: (i, 0)),
            pl.BlockSpec((8, co), lambda i: (0, 0)),
        ],
        out_shape=[
            jax.ShapeDtypeStruct((np_l, co), F32),
            jax.ShapeDtypeStruct((8, co), F32),
        ],
    )(aggp[0], aggp[1], degp[0], degp[1], lin)

    def body2(z_ref, st_ref, h_ref, res_ref, out_ref):
        mean = st_ref[0:1, :] * inv_n
        var = st_ref[1:2, :] * inv_n - mean * mean
        std = jnp.sqrt(jnp.maximum(var, 0.0))
        zn = (z_ref[...] - mean) / (std + 1e-5)
        out_ref[...] = jnp.maximum(zn, 0.0) + jnp.dot(
            h_ref[...], res_ref[...], preferred_element_type=F32)

    return pl.pallas_call(
        body2,
        grid=(nsteps,),
        in_specs=[
            pl.BlockSpec((NCH, co), lambda i: (i, 0)),
            pl.BlockSpec((8, co), lambda i: (0, 0)),
            pl.BlockSpec((NCH, ci), lambda i: (i, 0)),
            pl.BlockSpec((ci, co), lambda i: (0, 0)),
        ],
        out_specs=pl.BlockSpec((NCH, co), lambda i: (i, 0)),
        out_shape=jax.ShapeDtypeStruct((np_l, co), F32),
    )(z, st, h, res)


def _pool_call(a3):
    m = a3.shape[0]
    d = a3.shape[2]

    def body(a_ref, out_ref):
        out_ref[...] = (a_ref[:, 0, :] + a_ref[:, 1, :]) * 0.5

    return pl.pallas_call(
        body,
        grid=(m // NCH,),
        in_specs=[pl.BlockSpec((NCH, 2, d), lambda i: (i, 0, 0))],
        out_specs=pl.BlockSpec((NCH, d), lambda i: (i, 0)),
        out_shape=jax.ShapeDtypeStruct((m, d), F32),
    )(a3)


def _classifier_call(g0, g1, c0, c1, seq_emb, domain, ws, wq, wd,
                     wc1a, wc1b, wc1c, wc2):
    def body(g0_ref, g1_ref, c0_ref, c1_ref, se_ref, dom_ref, ws_ref,
             wq_ref, wd_ref, a_ref, b_ref, c_ref, w2_ref, out_ref):
        cnt = c0_ref[...][:, 0:1] + c1_ref[...][:, 0:1]
        g = (g0_ref[...] + g1_ref[...]) / jnp.maximum(cnt, 1.0)
        struct = jnp.dot(g, ws_ref[...], preferred_element_type=F32)
        seqf = jnp.dot(se_ref[...], wq_ref[...], preferred_element_type=F32)
        dom = dom_ref[...]
        mask = jnp.sum(dom, axis=1, keepdims=True) != 0.0
        domf = jnp.where(mask,
                         jnp.dot(dom, wd_ref[...], preferred_element_type=F32),
                         0.0)
        hid = (jnp.dot(struct, a_ref[...], preferred_element_type=F32)
               + jnp.dot(seqf, b_ref[...], preferred_element_type=F32)
               + jnp.dot(domf, c_ref[...], preferred_element_type=F32))
        mean = jnp.mean(hid, axis=0, keepdims=True)
        var = jnp.mean(hid * hid, axis=0, keepdims=True) - mean * mean
        std = jnp.sqrt(jnp.maximum(var, 0.0))
        hid = jnp.maximum((hid - mean) / (std + 1e-5), 0.0)
        out_ref[...] = jnp.dot(hid, w2_ref[...], preferred_element_type=F32)

    nc = wc2.shape[1]
    return pl.pallas_call(
        body,
        out_shape=jax.ShapeDtypeStruct((B, nc), F32),
    )(g0, g1, c0, c1, seq_emb, domain, ws, wq, wd, wc1a, wc1b, wc1c, wc2)


NW = 32
SCCH = 128


def _shift_idx2(idx_v, lvl, g):
    if lvl:
        for r in range(g):
            for t in range(SCCH // 16):
                sl = pl.ds(t * 16, 16)
                idx_v[r, sl] = jnp.right_shift(idx_v[r, sl], lvl)


def _grp(d):
    return max(1, min(8, 262144 // (SCCH * d * 4)))


def _sc_gather(table, idx2, lvl):
    d = table.shape[1]
    ep = idx2.shape[0] * SCCH
    per_w = ep // NW
    g = _grp(d)
    while g > 1 and (per_w // (SCCH * g)) % 2:
        g //= 2
    ngrp = per_w // (SCCH * g)
    assert ngrp % 2 == 0
    mesh = plsc.VectorSubcoreMesh(core_axis_name="c", subcore_axis_name="s")

    @functools.partial(
        pl.kernel,
        out_type=jax.ShapeDtypeStruct((ep, d), F32),
        mesh=mesh,
        compiler_params=pltpu.CompilerParams(use_tc_tiling_on_sc=False),
        scratch_types=[
            pltpu.VMEM((2 * g, SCCH), I32),
            pltpu.VMEM((2 * g * SCCH, d), F32),
            pltpu.SemaphoreType.DMA,
            pltpu.SemaphoreType.DMA,
        ],
    )
    def k(table_hbm, idx_hbm, out_hbm, idx_v, rows_v, sem_i, sem_g):
        wid = lax.axis_index("s") * 2 + lax.axis_index("c")
        base_w = wid * per_w
        row_w = base_w // SCCH

        def shift_rows(b):
            if lvl:
                for r in range(g):
                    for t in range(SCCH // 16):
                        sl = pl.ds(t * 16, 16)
                        idx_v[b * g + r, sl] = jnp.right_shift(
                            idx_v[b * g + r, sl], lvl)

        def start_idx(j, b):
            pltpu.async_copy(idx_hbm.at[pl.ds(row_w + j * g, g)],
                             idx_v.at[pl.ds(b * g, g)], sem_i)

        def wait_idx(b):
            pltpu.make_async_copy(idx_hbm.at[pl.ds(row_w, g)],
                                  idx_v.at[pl.ds(b * g, g)], sem_i).wait()

        def start_group(b):
            shift_rows(b)
            for r in range(g):
                pltpu.async_copy(
                    table_hbm.at[idx_v.at[b * g + r]],
                    rows_v.at[pl.ds((b * g + r) * SCCH, SCCH)], sem_g)

        def wait_group(b):
            for r in range(g):
                pltpu.make_async_copy(
                    table_hbm.at[idx_v.at[b * g + r]],
                    rows_v.at[pl.ds((b * g + r) * SCCH, SCCH)],
                    sem_g).wait()

        def write_out(j, b):
            pltpu.sync_copy(
                rows_v.at[pl.ds(b * g * SCCH, g * SCCH)],
                out_hbm.at[pl.ds(base_w + j * g * SCCH, g * SCCH)])

        pltpu.sync_copy(idx_hbm.at[pl.ds(row_w, g)],
                        idx_v.at[pl.ds(0, g)])
        start_group(0)
        npair = ngrp // 2

        def body(j2, carry):
            j = 2 * j2
            start_idx(j + 1, 1)
            wait_group(0)
            wait_idx(1)
            start_group(1)
            write_out(j, 0)

            @pl.when(j2 + 1 < npair)
            def _():
                start_idx(j + 2, 0)
                wait_group(1)
                wait_idx(0)
                start_group(0)
                write_out(j + 1, 1)

            @pl.when(j2 + 1 >= npair)
            def _():
                wait_group(1)
                write_out(j + 1, 1)

            return carry

        lax.fori_loop(0, npair, body, 0)

    return k(table, idx2)


def _sc_scatter(rows, idx2, lvl, np_l, d, ones_mode=False, col0=0,
                dfull=None):
    if dfull is None:
        dfull = d
    ep = idx2.shape[0] * SCCH
    per_w = ep // NW
    g = 2 if ones_mode else 1
    ngrp = per_w // (SCCH * g)
    rows_pt = np_l // 16
    zr = 8
    rbr = min(400, max(8, 4096 // d))
    while rows_pt % rbr:
        rbr //= 2
    mesh = plsc.VectorSubcoreMesh(core_axis_name="c", subcore_axis_name="s")

    scratch = [
        pltpu.VMEM((g, SCCH), I32),
        pltpu.VMEM((g * SCCH, d), F32),
        pltpu.VMEM((zr, d), F32),
        pltpu.VMEM((rbr, d), F32),
        pltpu.VMEM_SHARED((np_l, d), F32),
        pltpu.SemaphoreType.DMA,
    ]

    def body_common(rows_hbm, idx_hbm, out_hbm, idx_v, rows_v, zbuf, rbuf,
                    acc, sem):
        cid = lax.axis_index("c")
        sid = lax.axis_index("s")
        wid = sid * 2 + cid
        for r in range(rbr):
            for t in range(d // 16):
                rbuf[r, pl.ds(t * 16, 16)] = jnp.zeros((16,), F32)
        r0 = sid * rows_pt

        def zb(j, carry):
            pltpu.sync_copy(rbuf, acc.at[pl.ds(r0 + j * rbr, rbr)])
            return carry

        lax.fori_loop(0, rows_pt // rbr, zb, 0)

        if ones_mode:
            for r in range(g * SCCH):
                for t in range(d // 16):
                    rows_v[r, pl.ds(t * 16, 16)] = jnp.ones((16,), F32)

        plsc.subcore_barrier()
        base_w = wid * per_w

        def body(j, carry):
            base = base_w + j * (SCCH * g)
            pltpu.sync_copy(idx_hbm.at[pl.ds(base // SCCH, g)], idx_v)
            _shift_idx2(idx_v, lvl, g)
            if not ones_mode:
                if d == dfull:
                    pltpu.sync_copy(rows_hbm.at[pl.ds(base, SCCH * g)],
                                    rows_v)
                else:
                    pltpu.sync_copy(
                        rows_hbm.at[pl.ds(base, SCCH * g), pl.ds(col0, d)],
                        rows_v)
            for r in range(g):
                pltpu.sync_copy(rows_v.at[pl.ds(r * SCCH, SCCH)],
                                acc.at[idx_v.at[r]], add=True)
            return carry

        lax.fori_loop(0, ngrp, body, 0)
        plsc.subcore_barrier()

        def rb(j, carry):
            r = r0 + j * rbr
            pltpu.sync_copy(acc.at[pl.ds(r, rbr)], rbuf)
            pltpu.sync_copy(rbuf, out_hbm.at[cid].at[pl.ds(r, rbr)])
            return carry

        lax.fori_loop(0, rows_pt // rbr, rb, 0)

    if ones_mode:
        @functools.partial(
            pl.kernel,
            out_type=jax.ShapeDtypeStruct((2, np_l, d), F32),
            mesh=mesh, scratch_types=scratch,
            compiler_params=pltpu.CompilerParams(use_tc_tiling_on_sc=False),
        )
        def k1(idx_hbm, out_hbm, idx_v, rows_v, zbuf, rbuf, acc, sem):
            body_common(None, idx_hbm, out_hbm, idx_v, rows_v, zbuf, rbuf,
                        acc, sem)

        return k1(idx2)

    @functools.partial(
        pl.kernel,
        out_type=jax.ShapeDtypeStruct((2, np_l, d), F32),
        mesh=mesh, scratch_types=scratch,
        compiler_params=pltpu.CompilerParams(use_tc_tiling_on_sc=False),
    )
    def k2(rows_hbm, idx_hbm, out_hbm, idx_v, rows_v, zbuf, rbuf, acc, sem):
        body_common(rows_hbm, idx_hbm, out_hbm, idx_v, rows_v, zbuf, rbuf,
                    acc, sem)

    return k2(rows, idx2)


def _sc_gms(h_tab, kern, src2, dst2, lvl, np_l, col0=0):
    d = h_tab.shape[1]
    dk = kern.shape[1]
    ep = src2.shape[0] * SCCH
    per_w = ep // NW
    nch = per_w // SCCH
    gg = 2 if d <= 64 else 1
    nst = nch // gg
    rows_pt = np_l // 16
    zr = 8
    rbr = min(400, max(8, 4096 // d))
    while rows_pt % rbr:
        rbr //= 2
    mesh = plsc.VectorSubcoreMesh(core_axis_name="c", subcore_axis_name="s")

    @functools.partial(
        pl.kernel,
        out_type=jax.ShapeDtypeStruct((2, np_l, d), F32),
        mesh=mesh,
        compiler_params=pltpu.CompilerParams(use_tc_tiling_on_sc=False),
        scratch_types=[
            pltpu.VMEM((2 * gg, SCCH), I32),
            pltpu.VMEM((2 * gg, SCCH), I32),
            pltpu.VMEM((2 * gg * SCCH, d), F32),
            pltpu.VMEM((2 * gg * SCCH, d), F32),
            pltpu.VMEM((zr, d), F32),
            pltpu.VMEM((rbr, d), F32),
            pltpu.VMEM_SHARED((np_l, d), F32),
            pltpu.SemaphoreType.DMA,
            pltpu.SemaphoreType.DMA,
        ],
    )
    def k(h_hbm, kern_hbm, src_hbm, dst_hbm, out_hbm, siv, div, hv, kv,
          zbuf, rbuf, acc, sem_i, sem_g):
        cid = lax.axis_index("c")
        sid = lax.axis_index("s")
        wid = sid * 2 + cid
        for r in range(rbr):
            for t in range(d // 16):
                rbuf[r, pl.ds(t * 16, 16)] = jnp.zeros((16,), F32)
        r0 = sid * rows_pt

        def zb(j, carry):
            pltpu.sync_copy(rbuf, acc.at[pl.ds(r0 + j * rbr, rbr)])
            return carry

        lax.fori_loop(0, rows_pt // rbr, zb, 0)
        plsc.subcore_barrier()
        base_w = wid * per_w
        row_w = base_w // SCCH

        def shift_rows(b):
            if lvl:
                for r in range(gg):
                    for t in range(SCCH // 16):
                        sl = pl.ds(t * 16, 16)
                        rr = b * gg + r
                        siv[rr, sl] = jnp.right_shift(siv[rr, sl], lvl)
                        div[rr, sl] = jnp.right_shift(div[rr, sl], lvl)

        def kern_src(st, r):
            base = base_w + (st * gg + r) * SCCH
            if col0 == 0 and d == dk:
                return kern_hbm.at[pl.ds(base, SCCH)]
            return kern_hbm.at[pl.ds(base, SCCH), pl.ds(col0, d)]

        def start_group(st, b):
            shift_rows(b)
            for r in range(gg):
                rr = b * gg + r
                pltpu.async_copy(h_hbm.at[siv.at[rr]],
                                 hv.at[pl.ds(rr * SCCH, SCCH)], sem_g)
                pltpu.async_copy(kern_src(st, r),
                                 kv.at[pl.ds(rr * SCCH, SCCH)], sem_g)

        def wait_group(b):
            for r in range(gg):
                rr = b * gg + r
                pltpu.make_async_copy(
                    h_hbm.at[siv.at[rr]], hv.at[pl.ds(rr * SCCH, SCCH)],
                    sem_g).wait()
                pltpu.make_async_copy(
                    kern_src(0, r), kv.at[pl.ds(rr * SCCH, SCCH)],
                    sem_g).wait()

        def start_idx(st, b):
            pltpu.async_copy(src_hbm.at[pl.ds(row_w + st * gg, gg)],
                             siv.at[pl.ds(b * gg, gg)], sem_i)
            pltpu.async_copy(dst_hbm.at[pl.ds(row_w + st * gg, gg)],
                             div.at[pl.ds(b * gg, gg)], sem_i)

        def wait_idx(b):
            pltpu.make_async_copy(src_hbm.at[pl.ds(row_w, gg)],
                                  siv.at[pl.ds(b * gg, gg)], sem_i).wait()
            pltpu.make_async_copy(dst_hbm.at[pl.ds(row_w, gg)],
                                  div.at[pl.ds(b * gg, gg)], sem_i).wait()

        def compute_scatter(b):
            for r in range(gg):
                rr = b * gg + r

                def mulrow(q, carry):
                    row = rr * SCCH + q
                    for t in range(d // 16):
                        sl = pl.ds(t * 16, 16)
                        kv[row, sl] = kv[row, sl] * hv[row, sl]
                    return carry

                lax.fori_loop(0, SCCH, mulrow, 0, unroll=4)
                pltpu.sync_copy(kv.at[pl.ds(rr * SCCH, SCCH)],
                                acc.at[div.at[rr]], add=True)

        pltpu.sync_copy(src_hbm.at[pl.ds(row_w, gg)],
                        siv.at[pl.ds(0, gg)])
        pltpu.sync_copy(dst_hbm.at[pl.ds(row_w, gg)],
                        div.at[pl.ds(0, gg)])
        start_group(0, 0)
        npair = nst // 2

        def body(j2, carry):
            st = 2 * j2
            start_idx(st + 1, 1)
            wait_group(0)
            wait_idx(1)
            start_group(st + 1, 1)
            compute_scatter(0)

            @pl.when(j2 + 1 < npair)
            def _():
                start_idx(st + 2, 0)
                wait_group(1)
                wait_idx(0)
                start_group(st + 2, 0)
                compute_scatter(1)

            @pl.when(j2 + 1 >= npair)
            def _():
                wait_group(1)
                compute_scatter(1)

            return carry

        lax.fori_loop(0, npair, body, 0)
        plsc.subcore_barrier()

        def rb(j, carry):
            r = r0 + j * rbr
            pltpu.sync_copy(acc.at[pl.ds(r, rbr)], rbuf)
            pltpu.sync_copy(rbuf, out_hbm.at[cid].at[pl.ds(r, rbr)])
            return carry

        lax.fori_loop(0, rows_pt // rbr, rb, 0)

    return k(h_tab, kern, src2, dst2)


def _gms_dispatch(h, kern, src2, dst2, lvl, np_l):
    d = kern.shape[1]
    if np_l * d * 4 > 5_000_000:
        dh = d // 2
        parts = [_sc_gms(h[:, c:c + dh], kern, src2, dst2, lvl, np_l,
                         col0=c) for c in range(0, d, dh)]
        return jnp.concatenate(parts, axis=2)
    return _sc_gms(h, kern, src2, dst2, lvl, np_l)


def _sc_scatter_add(rows, idx, lvl, np_l):
    d = rows.shape[1]
    if np_l * d * 4 > 5_000_000 and d > 128:
        parts = [_sc_scatter(rows, idx, lvl, np_l, 128, col0=c, dfull=d)
                 for c in range(0, d, 128)]
        return jnp.concatenate(parts, axis=2)
    return _sc_scatter(rows, idx, lvl, np_l, d)


def _sc_scatter_ones(idx, lvl, np_l):
    return _sc_scatter(None, idx, lvl, np_l, 16, ones_mode=True)


def kernel(pos, seq, ori, domain, seq_emb, params, x, edge_index, batch):
    np0 = LVL_NP[0]
    src = edge_index[0].astype(I32)
    dst = edge_index[1].astype(I32)
    src_p = jnp.concatenate([src, jnp.zeros((EP - E,), I32)])
    dst_p = jnp.concatenate([dst, jnp.full((EP - E,), N0, I32)])
    s2 = src_p.reshape(EP, 1)
    d2 = dst_p.reshape(EP, 1)
    src_p = src_p.reshape(EP // 128, 128)
    dst_p = dst_p.reshape(EP // 128, 128)

    g_tab = jnp.concatenate(
        [pos, jnp.zeros((N0, 1), F32), ori.reshape(N0, 9),
         jnp.zeros((N0, 3), F32)], axis=1)
    g_tab = jnp.pad(g_tab, ((0, np0 - N0), (0, 0)))

    x2 = jnp.pad(x.astype(I32), (0, np0 - N0),
                 constant_values=31).reshape(np0, 1)
    emb32 = jnp.pad(params["emb"], ((0, 11), (0, 0)))
    h = _embed_call(x2, emb32, np0)

    g0 = g1 = c0 = c1 = None
    for lvl in range(4):
        n_l = LVL_N[lvl]
        np_l = LVL_NP[lvl]
        iA, iB = 2 * lvl, 2 * lvl + 1
        k1a = jnp.concatenate([params[f"b{iA}_k1"], jnp.zeros((3, 24), F32)])
        k1b = jnp.concatenate([params[f"b{iB}_k1"], jnp.zeros((3, 24), F32)])
        k2a = params[f"b{iA}_k2"]
        k2b = params[f"b{iB}_k2"]

        gs = _sc_gather(g_tab, src_p, lvl)
        gd = _sc_gather(g_tab, dst_p, lvl)
        degp = _sc_scatter_ones(dst_p, lvl, np_l)

        kern_a, kern_b = _kern_call(gs, gd, s2, d2,
                                    k1a, k2a, k1b, k2b, lvl)
        agg_a = _gms_dispatch(h, kern_a, src_p, dst_p, lvl, np_l)
        h = _node_update(agg_a, degp, h, params[f"b{iA}_lin"],
                         params[f"b{iA}_res"], n_l, np_l)

        agg_b = _gms_dispatch(h, kern_b, src_p, dst_p, lvl, np_l)
        h = _node_update(agg_b, degp, h, params[f"b{iB}_lin"],
                         params[f"b{iB}_res"], n_l, np_l)

        if lvl < 3:
            m = np_l // 2
            g_tab = _pool_call(g_tab.reshape(m, 2, 16))
            h = _pool_call(h.reshape(m, 2, h.shape[1]))

    batch_l = batch[::8].astype(I32)
    epg = 8192
    batch_p = jnp.pad(batch_l, (0, epg - LVL_N[3]),
                      constant_values=B).reshape(epg // 128, 128)
    h_p = jnp.pad(h, ((0, epg - LVL_NP[3]), (0, 0)))
    gpart = _sc_scatter_add(h_p, batch_p, 0, 128)
    cpart = _sc_scatter_ones(batch_p, 0, 128)

    wc1 = params["Wc1"]
    out = _classifier_call(
        gpart[0, :B], gpart[1, :B], cpart[0, :B], cpart[1, :B],
        seq_emb, domain, params["Ws"], params["Wq"], params["Wd"],
        wc1[0:256], wc1[256:512], wc1[512:768], params["Wc2"])
    return out
